# Initial kernel scaffold; baseline (speedup 1.0000x reference)
import jax
import jax.numpy as jnp
from jax.experimental import pallas as pl


def kernel(adjs, edges, start_day, end_day, W_gc0, b_gc0, W_gc1, b_gc1, W_ih, W_hh, b_ih, b_hh, bn_gamma, bn_beta, bn_rm, bn_rv, W_ep1, b_ep1, W_ep2, b_ep2, W_np1, b_np1, W_np2, b_np2, W_att, b_att):
    raise NotImplementedError("write your pallas kernel here")


# speedup vs baseline: 4.2152x; 4.2152x over previous
import functools

import jax
import jax.numpy as jnp
from jax import lax
from jax.experimental import pallas as pl
from jax.experimental.pallas import tpu as pltpu
from jax.experimental.pallas import tpu_sc as plsc

_N = 50000
_H = 64
_E = 800000
_EP = 100000
_NSC = 2
_NTILE = 16

_NT = ((_N + 1 + 127) // 128) * 128
_RPT = _NT // _NTILE
_TE = ((_E + _NTILE * 1024 - 1) // (_NTILE * 1024)) * 1024
_EPAD = _NTILE * _TE
_TEP = ((_EP + _NTILE * 256 - 1) // (_NTILE * 256)) * 256
_EPPAD = _NTILE * _TEP
_SENT = _N

_f32 = jnp.float32
_i32 = jnp.int32


def _mesh():
    return plsc.VectorSubcoreMesh(core_axis_name="c", subcore_axis_name="s")


_SC_PARAMS = pltpu.CompilerParams(use_tc_tiling_on_sc=False,
                                  needs_layout_passes=False)


def _zero_zbuf(zbuf):
    z = jnp.zeros((16,), _f32)

    def body(r, _):
        zbuf[r, pl.ds(0, 16)] = z
        zbuf[r, pl.ds(16, 16)] = z
        return 0

    lax.fori_loop(0, 128, body, 0)


def _zero_accum(accum, zbuf, s):
    base = s * _RPT
    nfull = _RPT // 128
    rem = _RPT % 128

    def body(k, _):
        pltpu.sync_copy(zbuf, accum.at[pl.ds(base + k * 128, 128)])
        return 0

    lax.fori_loop(0, nfull, body, 0)
    if rem:
        pltpu.sync_copy(zbuf.at[pl.ds(0, rem)],
                        accum.at[pl.ds(base + nfull * 128, rem)])


def _add_base(idx, nrows, cbase):
    for j in range(nrows):
        for t in range(8):
            sl = (j, pl.ds(t * 16, 16))
            idx[sl] = idx[sl] + cbase


def _spmm_body(table, dst2d, src2d, out, accum, idxs, idxd, rows, zbuf, sem):
    c = lax.axis_index("c")
    s = lax.axis_index("s")
    _zero_zbuf(zbuf)
    _zero_accum(accum, zbuf, s)
    plsc.subcore_barrier()
    cbase = c * _NT
    rows_per_tile = _TE // 128

    def outer(k, _):
        r0 = s * rows_per_tile + k * 8
        pltpu.sync_copy(src2d.at[pl.ds(r0, 8)], idxs)
        pltpu.sync_copy(dst2d.at[pl.ds(r0, 8)], idxd)
        _add_base(idxs, 8, cbase)
        for j in range(8):
            pltpu.async_copy(table.at[idxs.at[j]], rows, sem).wait()
            pltpu.sync_copy(rows, accum.at[idxd.at[j]], add=True)
        return 0

    lax.fori_loop(0, rows_per_tile // 8, outer, 0)
    plsc.subcore_barrier()
    wbase = s * _RPT
    pltpu.sync_copy(accum.at[pl.ds(wbase, _RPT)],
                    out.at[pl.ds(cbase + wbase, _RPT)])


_spmm = functools.partial(
    pl.kernel,
    out_type=jax.ShapeDtypeStruct((_NSC * _NT, 32), _f32),
    mesh=_mesh(),
    compiler_params=_SC_PARAMS,
    scratch_types=[
        pltpu.VMEM_SHARED((_NT, 32), _f32),
        pltpu.VMEM((8, 128), _i32),
        pltpu.VMEM((8, 128), _i32),
        pltpu.VMEM((128, 32), _f32),
        pltpu.VMEM((128, 32), _f32),
        pltpu.SemaphoreType.DMA,
    ],
)(_spmm_body)


def _aggr_body(s1h, s2h, table, dst2d, src2d, neigh_out, deg_out,
               nacc, dacc, a1buf, a2buf, idxd, idxs, rows, wbuf, degbuf,
               zbuf, sem):
    c = lax.axis_index("c")
    s = lax.axis_index("s")
    _zero_zbuf(zbuf)
    z16 = jnp.zeros((16,), _f32)
    for t in range(8):
        wbuf[pl.ds(t * 16, 16)] = z16
    _zero_accum(nacc, zbuf, s)
    base = s * _RPT
    nfull = _RPT // 128
    rem = _RPT % 128

    def zb(k, _):
        pltpu.sync_copy(wbuf, dacc.at[pl.ds(base + k * 128, 128)])
        return 0

    lax.fori_loop(0, nfull, zb, 0)
    if rem:
        pltpu.sync_copy(wbuf.at[pl.ds(0, rem)],
                        dacc.at[pl.ds(base + nfull * 128, rem)])
    plsc.subcore_barrier()
    cbase = c * _NT
    rows_per_tile = _TE // 128

    def outer(k, _):
        r0 = s * rows_per_tile + k * 8
        pltpu.sync_copy(dst2d.at[pl.ds(r0, 8)], idxd)
        pltpu.sync_copy(src2d.at[pl.ds(r0, 8)], idxs)
        for j in range(8):
            pltpu.async_copy(s1h.at[idxd.at[j]], a1buf, sem).wait()
            pltpu.async_copy(s2h.at[idxs.at[j]], a2buf, sem).wait()
            for t in range(8):
                d16 = idxd[j, pl.ds(t * 16, 16)]
                s16 = idxs[j, pl.ds(t * 16, 16)]
                a1 = a1buf[pl.ds(t * 16, 16)]
                a2 = a2buf[pl.ds(t * 16, 16)]
                w = 1.0 / (1.0 + jnp.exp(-(a1 + a2)))
                m = d16 != s16
                wbuf[pl.ds(t * 16, 16)] = jnp.where(m, w, 0.0)
                degbuf[j, pl.ds(t * 16, 16)] = jnp.where(m, 1.0, 0.0)
            pltpu.sync_copy(degbuf.at[j], dacc.at[idxd.at[j]], add=True)
            for t in range(8):
                ssl = (j, pl.ds(t * 16, 16))
                idxs[ssl] = idxs[ssl] + cbase
            pltpu.async_copy(table.at[idxs.at[j]], rows, sem).wait()

            def scale(r, _):
                bc = plsc.load_gather(wbuf, [jnp.full((16,), r, _i32)])
                rows[r, pl.ds(0, 16)] = rows[r, pl.ds(0, 16)] * bc
                rows[r, pl.ds(16, 16)] = rows[r, pl.ds(16, 16)] * bc
                return 0

            lax.fori_loop(0, 128, scale, 0)
            pltpu.sync_copy(rows, nacc.at[idxd.at[j]], add=True)
        return 0

    lax.fori_loop(0, rows_per_tile // 8, outer, 0)
    plsc.subcore_barrier()
    wbase = s * _RPT
    pltpu.sync_copy(nacc.at[pl.ds(wbase, _RPT)],
                    neigh_out.at[pl.ds(cbase + wbase, _RPT)])

    @pl.when(c == 0)
    def _():
        pltpu.sync_copy(dacc.at[pl.ds(wbase, _RPT)],
                        deg_out.at[pl.ds(wbase, _RPT)])


_aggr = functools.partial(
    pl.kernel,
    out_type=(jax.ShapeDtypeStruct((_NSC * _NT, 32), _f32),
              jax.ShapeDtypeStruct((_NT,), _f32)),
    mesh=_mesh(),
    compiler_params=_SC_PARAMS,
    scratch_types=[
        pltpu.VMEM_SHARED((_NT, 32), _f32),
        pltpu.VMEM_SHARED((_NT,), _f32),
        pltpu.VMEM((128,), _f32),
        pltpu.VMEM((128,), _f32),
        pltpu.VMEM((8, 128), _i32),
        pltpu.VMEM((8, 128), _i32),
        pltpu.VMEM((128, 32), _f32),
        pltpu.VMEM((128,), _f32),
        pltpu.VMEM((8, 128), _f32),
        pltpu.VMEM((128, 32), _f32),
        pltpu.SemaphoreType.DMA,
    ],
)(_aggr_body)


def _egather_body(table, e0_2d, e1_2d, out, idx, rows, sem):
    c = lax.axis_index("c")
    s = lax.axis_index("s")
    cbase = c * _NT
    rows_per_tile = _TEP // 128

    def outer(k, _):
        r0 = s * rows_per_tile + k * 2
        for j in range(2):
            earr = e0_2d if j == 0 else e1_2d
            pltpu.sync_copy(earr.at[pl.ds(r0, 2)], idx)
            _add_base(idx, 2, cbase)
            for t in range(2):
                pltpu.async_copy(table.at[idx.at[t]], rows, sem).wait()
                sec = (2 * j + c) * _EPPAD
                pltpu.sync_copy(
                    rows, out.at[pl.ds(sec + (r0 + t) * 128, 128)])
        return 0

    lax.fori_loop(0, rows_per_tile // 2, outer, 0)


_egather = functools.partial(
    pl.kernel,
    out_type=jax.ShapeDtypeStruct((4 * _EPPAD, 32), _f32),
    mesh=_mesh(),
    compiler_params=_SC_PARAMS,
    scratch_types=[
        pltpu.VMEM((2, 128), _i32),
        pltpu.VMEM((128, 32), _f32),
        pltpu.SemaphoreType.DMA,
    ],
)(_egather_body)


def _relu_mm_body(x2, b, w, o_ref):
    x = jnp.concatenate([x2[0], x2[1]], axis=1) + b[...]
    x = jnp.maximum(x, 0.0)
    sup = jnp.dot(x, w[...], preferred_element_type=_f32)
    o_ref[0] = sup[:, :32]
    o_ref[1] = sup[:, 32:]


def _tc_relu_mm(out1, b_gc0, W_gc1):
    bn = 1024
    grid = (_NT + bn - 1) // bn
    return pl.pallas_call(
        _relu_mm_body,
        grid=(grid,),
        in_specs=[
            pl.BlockSpec((2, bn, 32), lambda i: (0, i, 0)),
            pl.BlockSpec((1, _H), lambda i: (0, 0)),
            pl.BlockSpec((_H, _H), lambda i: (0, 0)),
        ],
        out_specs=pl.BlockSpec((2, bn, 32), lambda i: (0, i, 0)),
        out_shape=jax.ShapeDtypeStruct((2, _NT, 32), _f32),
    )(out1, b_gc0.reshape(1, _H), W_gc1)


def _gru_body(x2, b1, h, wih, whh, bih, bhh, o_ref):
    x = jnp.concatenate([x2[0], x2[1]], axis=1) + b1[...]
    gi = jnp.dot(x, wih[...], preferred_element_type=_f32) + bih[...]
    gh = jnp.dot(h[...], whh[...], preferred_element_type=_f32) + bhh[...]
    r = jax.nn.sigmoid(gi[:, :_H] + gh[:, :_H])
    z = jax.nn.sigmoid(gi[:, _H:2 * _H] + gh[:, _H:2 * _H])
    n = jnp.tanh(gi[:, 2 * _H:] + r * gh[:, 2 * _H:])
    o_ref[...] = (1.0 - z) * n + z * h[...]


def _tc_gru(out2, b_gc1, h, W_ihT, W_hhT, b_ih, b_hh):
    bn = 1000
    grid = _N // bn
    return pl.pallas_call(
        _gru_body,
        grid=(grid,),
        in_specs=[
            pl.BlockSpec((2, bn, 32), lambda i: (0, i, 0)),
            pl.BlockSpec((1, _H), lambda i: (0, 0)),
            pl.BlockSpec((bn, _H), lambda i: (i, 0)),
            pl.BlockSpec((_H, 3 * _H), lambda i: (0, 0)),
            pl.BlockSpec((_H, 3 * _H), lambda i: (0, 0)),
            pl.BlockSpec((1, 3 * _H), lambda i: (0, 0)),
            pl.BlockSpec((1, 3 * _H), lambda i: (0, 0)),
        ],
        out_specs=pl.BlockSpec((bn, _H), lambda i: (i, 0)),
        out_shape=jax.ShapeDtypeStruct((_N, _H), _f32),
    )(out2, b_gc1.reshape(1, _H), h, W_ihT, W_hhT,
      b_ih.reshape(1, 3 * _H), b_hh.reshape(1, 3 * _H))


def _post_body(h, scale, shift, wa, bvec, o_emb, o_emb2, o_s12):
    emb = h[...] * scale[...] + shift[...]
    o_emb[...] = emb
    o_emb2[0] = emb[:, :32]
    o_emb2[1] = emb[:, 32:]
    o_s12[...] = jnp.dot(emb, wa[...], preferred_element_type=_f32) + bvec[...]


def _tc_post(h, scale, shift, wa, bvec):
    bn = 1024
    grid = (_NT + bn - 1) // bn
    return pl.pallas_call(
        _post_body,
        grid=(grid,),
        in_specs=[
            pl.BlockSpec((bn, _H), lambda i: (i, 0)),
            pl.BlockSpec((1, _H), lambda i: (0, 0)),
            pl.BlockSpec((1, _H), lambda i: (0, 0)),
            pl.BlockSpec((_H, 2), lambda i: (0, 0)),
            pl.BlockSpec((1, 2), lambda i: (0, 0)),
        ],
        out_specs=[
            pl.BlockSpec((bn, _H), lambda i: (i, 0)),
            pl.BlockSpec((2, bn, 32), lambda i: (0, i, 0)),
            pl.BlockSpec((bn, 2), lambda i: (i, 0)),
        ],
        out_shape=[
            jax.ShapeDtypeStruct((_NT, _H), _f32),
            jax.ShapeDtypeStruct((2, _NT, 32), _f32),
            jax.ShapeDtypeStruct((_NT, 2), _f32),
        ],
    )(h, scale, shift, wa, bvec)


def _log_softmax2(lg):
    m = jnp.max(lg, axis=1, keepdims=True)
    ls = m + jnp.log(jnp.sum(jnp.exp(lg - m), axis=1, keepdims=True))
    return lg - ls


def _edge_mlp_body(nf0, nf1, ns0, ns1, w1, b1, w2, b2, o_ref):
    pe = (jnp.dot(nf0[...], w1[0:32, :], preferred_element_type=_f32)
          + jnp.dot(nf1[...], w1[32:64, :], preferred_element_type=_f32)
          + jnp.dot(ns0[...], w1[64:96, :], preferred_element_type=_f32)
          + jnp.dot(ns1[...], w1[96:128, :], preferred_element_type=_f32)
          + b1[...])
    pe = jnp.maximum(pe, 0.0)
    lg = jnp.dot(pe, w2[...], preferred_element_type=_f32) + b2[...]
    o_ref[...] = _log_softmax2(lg)


def _tc_edge_mlp(eg, W_ep1, b_ep1, W_ep2, b_ep2):
    bn = 512
    grid = (_EP + bn - 1) // bn
    nh = W_ep1.shape[1]
    sec = _EPPAD // bn

    def spec(k):
        return pl.BlockSpec((bn, 32), lambda i, k=k: (k * sec + i, 0))

    return pl.pallas_call(
        _edge_mlp_body,
        grid=(grid,),
        in_specs=[
            spec(0), spec(1), spec(2), spec(3),
            pl.BlockSpec((2 * _H, nh), lambda i: (0, 0)),
            pl.BlockSpec((1, nh), lambda i: (0, 0)),
            pl.BlockSpec((nh, 2), lambda i: (0, 0)),
            pl.BlockSpec((1, 2), lambda i: (0, 0)),
        ],
        out_specs=pl.BlockSpec((bn, 2), lambda i: (i, 0)),
        out_shape=jax.ShapeDtypeStruct((_EP, 2), _f32),
    )(eg, eg, eg, eg, W_ep1, b_ep1.reshape(1, nh), W_ep2,
      b_ep2.reshape(1, 2))


def _node_mlp_body(emb, n2, deg, w1, b1, w2, b2, o_ref):
    nb = jnp.concatenate([n2[0], n2[1]], axis=1)
    nb = nb / jnp.maximum(deg[...], 1.0)
    pn = (jnp.dot(emb[...], w1[0:_H, :], preferred_element_type=_f32)
          + jnp.dot(nb, w1[_H:2 * _H, :], preferred_element_type=_f32)
          + b1[...])
    pn = jnp.maximum(pn, 0.0)
    lg = jnp.dot(pn, w2[...], preferred_element_type=_f32) + b2[...]
    o_ref[...] = _log_softmax2(lg)


def _tc_node_mlp(emb, neigh, deg, W_np1, b_np1, W_np2, b_np2):
    bn = 1000
    grid = _N // bn
    nh = W_np1.shape[1]
    return pl.pallas_call(
        _node_mlp_body,
        grid=(grid,),
        in_specs=[
            pl.BlockSpec((bn, _H), lambda i: (i, 0)),
            pl.BlockSpec((2, bn, 32), lambda i: (0, i, 0)),
            pl.BlockSpec((bn, 1), lambda i: (i, 0)),
            pl.BlockSpec((2 * _H, nh), lambda i: (0, 0)),
            pl.BlockSpec((1, nh), lambda i: (0, 0)),
            pl.BlockSpec((nh, 2), lambda i: (0, 0)),
            pl.BlockSpec((1, 2), lambda i: (0, 0)),
        ],
        out_specs=pl.BlockSpec((bn, 2), lambda i: (i, 0)),
        out_shape=jax.ShapeDtypeStruct((_N, 2), _f32),
    )(emb, neigh, deg, W_np1, b_np1.reshape(1, nh), W_np2,
      b_np2.reshape(1, 2))


def kernel(adjs, edges, start_day, end_day, W_gc0, b_gc0, W_gc1, b_gc1,
           W_ih, W_hh, b_ih, b_hh, bn_gamma, bn_beta, bn_rm, bn_rv,
           W_ep1, b_ep1, W_ep2, b_ep2, W_np1, b_np1, W_np2, b_np2,
           W_att, b_att):
    adjs = adjs.astype(_i32)
    edges = edges.astype(_i32)
    nsnap = adjs.shape[0]

    def prep_adj(i):
        a = lax.dynamic_index_in_dim(adjs, i, 0, keepdims=False)
        pad = jnp.full((2, _EPAD - _E), _SENT, _i32)
        a = jnp.concatenate([a, pad], axis=1)
        return (a[0].reshape(_EPAD // 128, 128),
                a[1].reshape(_EPAD // 128, 128))

    T0 = jnp.zeros((_NSC * _NT, 32), _f32)
    T0 = T0.at[0:_N].set(W_gc0[:, :32]).at[_NT:_NT + _N].set(W_gc0[:, 32:])

    h = jnp.zeros((_N, _H), _f32)
    W_ihT = W_ih.T
    W_hhT = W_hh.T
    for i in range(nsnap - 1):
        dst, src = prep_adj(start_day + i)
        out1 = _spmm(T0, dst, src)
        sup2 = _tc_relu_mm(out1.reshape(2, _NT, 32), b_gc0, W_gc1)
        out2 = _spmm(sup2.reshape(_NSC * _NT, 32), dst, src)
        h = _tc_gru(out2.reshape(2, _NT, 32), b_gc1, h, W_ihT, W_hhT,
                    b_ih, b_hh)

    scale = (bn_gamma / jnp.sqrt(bn_rv + 1e-5)).reshape(1, _H)
    shift = (bn_beta.reshape(1, _H) - bn_rm.reshape(1, _H) * scale)
    wa = jnp.stack([W_att[:_H, 0], W_att[_H:, 0]], axis=1)
    bvec = jnp.stack([b_att[0], jnp.zeros((), _f32)]).reshape(1, 2)
    emb, emb2, s12 = _tc_post(h, scale, shift, wa, bvec)
    embT = emb2.reshape(_NSC * _NT, 32)

    epad = jnp.concatenate(
        [edges, jnp.zeros((2, _EPPAD - _EP), _i32)], axis=1)
    e0 = epad[0].reshape(_EPPAD // 128, 128)
    e1 = epad[1].reshape(_EPPAD // 128, 128)
    eg = _egather(embT, e0, e1)
    pred_edges = _tc_edge_mlp(eg, W_ep1, b_ep1, W_ep2, b_ep2)

    dst4, src4 = prep_adj(end_day + 1)
    s1 = s12[:, 0]
    s2 = s12[:, 1]
    neigh, deg = _aggr(s1, s2, embT, dst4, src4)
    pred_nodes = _tc_node_mlp(emb, neigh.reshape(2, _NT, 32),
                              deg.reshape(_NT, 1), W_np1, b_np1,
                              W_np2, b_np2)
    return (pred_edges, pred_nodes)


# speedup vs baseline: 5.3112x; 1.2600x over previous
import functools

import jax
import jax.numpy as jnp
from jax import lax
from jax.experimental import pallas as pl
from jax.experimental.pallas import tpu as pltpu
from jax.experimental.pallas import tpu_sc as plsc

_N = 50000
_H = 64
_E = 800000
_EP = 100000
_NSC = 2
_NTILE = 16

_NT = ((_N + 1 + 127) // 128) * 128
_RPT = _NT // _NTILE
_TE = ((_E + _NTILE * 1024 - 1) // (_NTILE * 1024)) * 1024
_EPAD = _NTILE * _TE
_TEP = ((_EP + _NTILE * 256 - 1) // (_NTILE * 256)) * 256
_EPPAD = _NTILE * _TEP
_SENT = _N

_f32 = jnp.float32
_i32 = jnp.int32


def _mesh():
    return plsc.VectorSubcoreMesh(core_axis_name="c", subcore_axis_name="s")


_SC_PARAMS = pltpu.CompilerParams(use_tc_tiling_on_sc=False,
                                  needs_layout_passes=False)


def _zero_zbuf(zbuf):
    z = jnp.zeros((16,), _f32)

    def body(r, _):
        zbuf[r, pl.ds(0, 16)] = z
        zbuf[r, pl.ds(16, 16)] = z
        return 0

    lax.fori_loop(0, 128, body, 0)


def _zero_accum(accum, zbuf, s):
    base = s * _RPT
    nfull = _RPT // 128
    rem = _RPT % 128

    def body(k, _):
        pltpu.sync_copy(zbuf, accum.at[pl.ds(base + k * 128, 128)])
        return 0

    lax.fori_loop(0, nfull, body, 0)
    if rem:
        pltpu.sync_copy(zbuf.at[pl.ds(0, rem)],
                        accum.at[pl.ds(base + nfull * 128, rem)])


def _add_base(idx, nrows, cbase):
    for j in range(nrows):
        for t in range(8):
            sl = (j, pl.ds(t * 16, 16))
            idx[sl] = idx[sl] + cbase


def _spmm_body(table, dst2d, src2d, out, accum, idxs, idxd, rows, zbuf,
               sem_g, sem_s):
    c = lax.axis_index("c")
    s = lax.axis_index("s")
    _zero_zbuf(zbuf)
    _zero_accum(accum, zbuf, s)
    plsc.subcore_barrier()
    cbase = c * _NT
    rows_per_tile = _TE // 128

    def outer(k, _):
        r0 = s * rows_per_tile + k * 8
        pltpu.sync_copy(src2d.at[pl.ds(r0, 8)], idxs)
        pltpu.sync_copy(dst2d.at[pl.ds(r0, 8)], idxd)
        _add_base(idxs, 8, cbase)
        scat = [None, None]
        g = pltpu.async_copy(table.at[idxs.at[0]], rows.at[0], sem_g)
        for j in range(8):
            b = j & 1
            nb = 1 - b
            if j < 7:
                if scat[nb] is not None:
                    scat[nb].wait()
                g_next = pltpu.async_copy(table.at[idxs.at[j + 1]],
                                          rows.at[nb], sem_g)
            g.wait()
            scat[b] = pltpu.async_copy(rows.at[b], accum.at[idxd.at[j]],
                                       sem_s, add=True)
            if j < 7:
                g = g_next
        scat[0].wait()
        scat[1].wait()
        return 0

    lax.fori_loop(0, rows_per_tile // 8, outer, 0)
    plsc.subcore_barrier()
    wbase = s * _RPT
    pltpu.sync_copy(accum.at[pl.ds(wbase, _RPT)],
                    out.at[pl.ds(cbase + wbase, _RPT)])


_spmm = functools.partial(
    pl.kernel,
    out_type=jax.ShapeDtypeStruct((_NSC * _NT, 32), _f32),
    mesh=_mesh(),
    compiler_params=_SC_PARAMS,
    scratch_types=[
        pltpu.VMEM_SHARED((_NT, 32), _f32),
        pltpu.VMEM((8, 128), _i32),
        pltpu.VMEM((8, 128), _i32),
        pltpu.VMEM((2, 128, 32), _f32),
        pltpu.VMEM((128, 32), _f32),
        pltpu.SemaphoreType.DMA,
        pltpu.SemaphoreType.DMA,
    ],
)(_spmm_body)


def _aggr_body(s1h, s2h, table, dst2d, src2d, neigh_out, deg_out,
               nacc, dacc, a1buf, a2buf, idxd, idxs, rows, wbuf, degbuf,
               zbuf, sem):
    c = lax.axis_index("c")
    s = lax.axis_index("s")
    _zero_zbuf(zbuf)
    z16 = jnp.zeros((16,), _f32)
    for t in range(8):
        wbuf[pl.ds(t * 16, 16)] = z16
    _zero_accum(nacc, zbuf, s)
    base = s * _RPT
    nfull = _RPT // 128
    rem = _RPT % 128

    def zb(k, _):
        pltpu.sync_copy(wbuf, dacc.at[pl.ds(base + k * 128, 128)])
        return 0

    lax.fori_loop(0, nfull, zb, 0)
    if rem:
        pltpu.sync_copy(wbuf.at[pl.ds(0, rem)],
                        dacc.at[pl.ds(base + nfull * 128, rem)])
    plsc.subcore_barrier()
    cbase = c * _NT
    rows_per_tile = _TE // 128

    def outer(k, _):
        r0 = s * rows_per_tile + k * 8
        pltpu.sync_copy(dst2d.at[pl.ds(r0, 8)], idxd)
        pltpu.sync_copy(src2d.at[pl.ds(r0, 8)], idxs)
        for j in range(8):
            pltpu.async_copy(s1h.at[idxd.at[j]], a1buf, sem).wait()
            pltpu.async_copy(s2h.at[idxs.at[j]], a2buf, sem).wait()
            for t in range(8):
                d16 = idxd[j, pl.ds(t * 16, 16)]
                s16 = idxs[j, pl.ds(t * 16, 16)]
                a1 = a1buf[pl.ds(t * 16, 16)]
                a2 = a2buf[pl.ds(t * 16, 16)]
                w = 1.0 / (1.0 + jnp.exp(-(a1 + a2)))
                m = d16 != s16
                wbuf[pl.ds(t * 16, 16)] = jnp.where(m, w, 0.0)
                degbuf[j, pl.ds(t * 16, 16)] = jnp.where(m, 1.0, 0.0)
            pltpu.sync_copy(degbuf.at[j], dacc.at[idxd.at[j]], add=True)
            for t in range(8):
                ssl = (j, pl.ds(t * 16, 16))
                idxs[ssl] = idxs[ssl] + cbase
            pltpu.async_copy(table.at[idxs.at[j]], rows, sem).wait()

            def scale(r, _):
                bc = plsc.load_gather(wbuf, [jnp.full((16,), r, _i32)])
                rows[r, pl.ds(0, 16)] = rows[r, pl.ds(0, 16)] * bc
                rows[r, pl.ds(16, 16)] = rows[r, pl.ds(16, 16)] * bc
                return 0

            lax.fori_loop(0, 128, scale, 0)
            pltpu.sync_copy(rows, nacc.at[idxd.at[j]], add=True)
        return 0

    lax.fori_loop(0, rows_per_tile // 8, outer, 0)
    plsc.subcore_barrier()
    wbase = s * _RPT
    pltpu.sync_copy(nacc.at[pl.ds(wbase, _RPT)],
                    neigh_out.at[pl.ds(cbase + wbase, _RPT)])

    @pl.when(c == 0)
    def _():
        pltpu.sync_copy(dacc.at[pl.ds(wbase, _RPT)],
                        deg_out.at[pl.ds(wbase, _RPT)])


_aggr = functools.partial(
    pl.kernel,
    out_type=(jax.ShapeDtypeStruct((_NSC * _NT, 32), _f32),
              jax.ShapeDtypeStruct((_NT,), _f32)),
    mesh=_mesh(),
    compiler_params=_SC_PARAMS,
    scratch_types=[
        pltpu.VMEM_SHARED((_NT, 32), _f32),
        pltpu.VMEM_SHARED((_NT,), _f32),
        pltpu.VMEM((128,), _f32),
        pltpu.VMEM((128,), _f32),
        pltpu.VMEM((8, 128), _i32),
        pltpu.VMEM((8, 128), _i32),
        pltpu.VMEM((128, 32), _f32),
        pltpu.VMEM((128,), _f32),
        pltpu.VMEM((8, 128), _f32),
        pltpu.VMEM((128, 32), _f32),
        pltpu.SemaphoreType.DMA,
    ],
)(_aggr_body)


def _egather_body(table, e0_2d, e1_2d, out, idx, rows, sem):
    c = lax.axis_index("c")
    s = lax.axis_index("s")
    cbase = c * _NT
    rows_per_tile = _TEP // 128

    def outer(k, _):
        r0 = s * rows_per_tile + k * 2
        for j in range(2):
            earr = e0_2d if j == 0 else e1_2d
            pltpu.sync_copy(earr.at[pl.ds(r0, 2)], idx)
            _add_base(idx, 2, cbase)
            for t in range(2):
                pltpu.async_copy(table.at[idx.at[t]], rows, sem).wait()
                sec = (2 * j + c) * _EPPAD
                pltpu.sync_copy(
                    rows, out.at[pl.ds(sec + (r0 + t) * 128, 128)])
        return 0

    lax.fori_loop(0, rows_per_tile // 2, outer, 0)


_egather = functools.partial(
    pl.kernel,
    out_type=jax.ShapeDtypeStruct((4 * _EPPAD, 32), _f32),
    mesh=_mesh(),
    compiler_params=_SC_PARAMS,
    scratch_types=[
        pltpu.VMEM((2, 128), _i32),
        pltpu.VMEM((128, 32), _f32),
        pltpu.SemaphoreType.DMA,
    ],
)(_egather_body)


def _relu_mm_body(x2, b, w, o_ref):
    x = jnp.concatenate([x2[0], x2[1]], axis=1) + b[...]
    x = jnp.maximum(x, 0.0)
    sup = jnp.dot(x, w[...], preferred_element_type=_f32)
    o_ref[0] = sup[:, :32]
    o_ref[1] = sup[:, 32:]


def _tc_relu_mm(out1, b_gc0, W_gc1):
    bn = 1024
    grid = (_NT + bn - 1) // bn
    return pl.pallas_call(
        _relu_mm_body,
        grid=(grid,),
        in_specs=[
            pl.BlockSpec((2, bn, 32), lambda i: (0, i, 0)),
            pl.BlockSpec((1, _H), lambda i: (0, 0)),
            pl.BlockSpec((_H, _H), lambda i: (0, 0)),
        ],
        out_specs=pl.BlockSpec((2, bn, 32), lambda i: (0, i, 0)),
        out_shape=jax.ShapeDtypeStruct((2, _NT, 32), _f32),
    )(out1, b_gc0.reshape(1, _H), W_gc1)


def _gru_body(x2, b1, h, wih, whh, bih, bhh, o_ref):
    x = jnp.concatenate([x2[0], x2[1]], axis=1) + b1[...]
    gi = jnp.dot(x, wih[...], preferred_element_type=_f32) + bih[...]
    gh = jnp.dot(h[...], whh[...], preferred_element_type=_f32) + bhh[...]
    r = jax.nn.sigmoid(gi[:, :_H] + gh[:, :_H])
    z = jax.nn.sigmoid(gi[:, _H:2 * _H] + gh[:, _H:2 * _H])
    n = jnp.tanh(gi[:, 2 * _H:] + r * gh[:, 2 * _H:])
    o_ref[...] = (1.0 - z) * n + z * h[...]


def _tc_gru(out2, b_gc1, h, W_ihT, W_hhT, b_ih, b_hh):
    bn = 1000
    grid = _N // bn
    return pl.pallas_call(
        _gru_body,
        grid=(grid,),
        in_specs=[
            pl.BlockSpec((2, bn, 32), lambda i: (0, i, 0)),
            pl.BlockSpec((1, _H), lambda i: (0, 0)),
            pl.BlockSpec((bn, _H), lambda i: (i, 0)),
            pl.BlockSpec((_H, 3 * _H), lambda i: (0, 0)),
            pl.BlockSpec((_H, 3 * _H), lambda i: (0, 0)),
            pl.BlockSpec((1, 3 * _H), lambda i: (0, 0)),
            pl.BlockSpec((1, 3 * _H), lambda i: (0, 0)),
        ],
        out_specs=pl.BlockSpec((bn, _H), lambda i: (i, 0)),
        out_shape=jax.ShapeDtypeStruct((_N, _H), _f32),
    )(out2, b_gc1.reshape(1, _H), h, W_ihT, W_hhT,
      b_ih.reshape(1, 3 * _H), b_hh.reshape(1, 3 * _H))


def _post_body(h, scale, shift, wa, bvec, o_emb, o_emb2, o_s12):
    emb = h[...] * scale[...] + shift[...]
    o_emb[...] = emb
    o_emb2[0] = emb[:, :32]
    o_emb2[1] = emb[:, 32:]
    o_s12[...] = jnp.dot(emb, wa[...], preferred_element_type=_f32) + bvec[...]


def _tc_post(h, scale, shift, wa, bvec):
    bn = 1024
    grid = (_NT + bn - 1) // bn
    return pl.pallas_call(
        _post_body,
        grid=(grid,),
        in_specs=[
            pl.BlockSpec((bn, _H), lambda i: (i, 0)),
            pl.BlockSpec((1, _H), lambda i: (0, 0)),
            pl.BlockSpec((1, _H), lambda i: (0, 0)),
            pl.BlockSpec((_H, 2), lambda i: (0, 0)),
            pl.BlockSpec((1, 2), lambda i: (0, 0)),
        ],
        out_specs=[
            pl.BlockSpec((bn, _H), lambda i: (i, 0)),
            pl.BlockSpec((2, bn, 32), lambda i: (0, i, 0)),
            pl.BlockSpec((bn, 2), lambda i: (i, 0)),
        ],
        out_shape=[
            jax.ShapeDtypeStruct((_NT, _H), _f32),
            jax.ShapeDtypeStruct((2, _NT, 32), _f32),
            jax.ShapeDtypeStruct((_NT, 2), _f32),
        ],
    )(h, scale, shift, wa, bvec)


def _log_softmax2(lg):
    m = jnp.max(lg, axis=1, keepdims=True)
    ls = m + jnp.log(jnp.sum(jnp.exp(lg - m), axis=1, keepdims=True))
    return lg - ls


def _edge_mlp_body(nf0, nf1, ns0, ns1, w1, b1, w2, b2, o_ref):
    pe = (jnp.dot(nf0[...], w1[0:32, :], preferred_element_type=_f32)
          + jnp.dot(nf1[...], w1[32:64, :], preferred_element_type=_f32)
          + jnp.dot(ns0[...], w1[64:96, :], preferred_element_type=_f32)
          + jnp.dot(ns1[...], w1[96:128, :], preferred_element_type=_f32)
          + b1[...])
    pe = jnp.maximum(pe, 0.0)
    lg = jnp.dot(pe, w2[...], preferred_element_type=_f32) + b2[...]
    o_ref[...] = _log_softmax2(lg)


def _tc_edge_mlp(eg, W_ep1, b_ep1, W_ep2, b_ep2):
    bn = 512
    grid = (_EP + bn - 1) // bn
    nh = W_ep1.shape[1]
    sec = _EPPAD // bn

    def spec(k):
        return pl.BlockSpec((bn, 32), lambda i, k=k: (k * sec + i, 0))

    return pl.pallas_call(
        _edge_mlp_body,
        grid=(grid,),
        in_specs=[
            spec(0), spec(1), spec(2), spec(3),
            pl.BlockSpec((2 * _H, nh), lambda i: (0, 0)),
            pl.BlockSpec((1, nh), lambda i: (0, 0)),
            pl.BlockSpec((nh, 2), lambda i: (0, 0)),
            pl.BlockSpec((1, 2), lambda i: (0, 0)),
        ],
        out_specs=pl.BlockSpec((bn, 2), lambda i: (i, 0)),
        out_shape=jax.ShapeDtypeStruct((_EP, 2), _f32),
    )(eg, eg, eg, eg, W_ep1, b_ep1.reshape(1, nh), W_ep2,
      b_ep2.reshape(1, 2))


def _node_mlp_body(emb, n2, deg, w1, b1, w2, b2, o_ref):
    nb = jnp.concatenate([n2[0], n2[1]], axis=1)
    nb = nb / jnp.maximum(deg[...], 1.0)
    pn = (jnp.dot(emb[...], w1[0:_H, :], preferred_element_type=_f32)
          + jnp.dot(nb, w1[_H:2 * _H, :], preferred_element_type=_f32)
          + b1[...])
    pn = jnp.maximum(pn, 0.0)
    lg = jnp.dot(pn, w2[...], preferred_element_type=_f32) + b2[...]
    o_ref[...] = _log_softmax2(lg)


def _tc_node_mlp(emb, neigh, deg, W_np1, b_np1, W_np2, b_np2):
    bn = 1000
    grid = _N // bn
    nh = W_np1.shape[1]
    return pl.pallas_call(
        _node_mlp_body,
        grid=(grid,),
        in_specs=[
            pl.BlockSpec((bn, _H), lambda i: (i, 0)),
            pl.BlockSpec((2, bn, 32), lambda i: (0, i, 0)),
            pl.BlockSpec((bn, 1), lambda i: (i, 0)),
            pl.BlockSpec((2 * _H, nh), lambda i: (0, 0)),
            pl.BlockSpec((1, nh), lambda i: (0, 0)),
            pl.BlockSpec((nh, 2), lambda i: (0, 0)),
            pl.BlockSpec((1, 2), lambda i: (0, 0)),
        ],
        out_specs=pl.BlockSpec((bn, 2), lambda i: (i, 0)),
        out_shape=jax.ShapeDtypeStruct((_N, 2), _f32),
    )(emb, neigh, deg, W_np1, b_np1.reshape(1, nh), W_np2,
      b_np2.reshape(1, 2))


def kernel(adjs, edges, start_day, end_day, W_gc0, b_gc0, W_gc1, b_gc1,
           W_ih, W_hh, b_ih, b_hh, bn_gamma, bn_beta, bn_rm, bn_rv,
           W_ep1, b_ep1, W_ep2, b_ep2, W_np1, b_np1, W_np2, b_np2,
           W_att, b_att):
    adjs = adjs.astype(_i32)
    edges = edges.astype(_i32)
    nsnap = adjs.shape[0]

    def prep_adj(i):
        a = lax.dynamic_index_in_dim(adjs, i, 0, keepdims=False)
        pad = jnp.full((2, _EPAD - _E), _SENT, _i32)
        a = jnp.concatenate([a, pad], axis=1)
        return (a[0].reshape(_EPAD // 128, 128),
                a[1].reshape(_EPAD // 128, 128))

    T0 = jnp.zeros((_NSC * _NT, 32), _f32)
    T0 = T0.at[0:_N].set(W_gc0[:, :32]).at[_NT:_NT + _N].set(W_gc0[:, 32:])

    h = jnp.zeros((_N, _H), _f32)
    W_ihT = W_ih.T
    W_hhT = W_hh.T
    for i in range(nsnap - 1):
        dst, src = prep_adj(start_day + i)
        out1 = _spmm(T0, dst, src)
        sup2 = _tc_relu_mm(out1.reshape(2, _NT, 32), b_gc0, W_gc1)
        out2 = _spmm(sup2.reshape(_NSC * _NT, 32), dst, src)
        h = _tc_gru(out2.reshape(2, _NT, 32), b_gc1, h, W_ihT, W_hhT,
                    b_ih, b_hh)

    scale = (bn_gamma / jnp.sqrt(bn_rv + 1e-5)).reshape(1, _H)
    shift = (bn_beta.reshape(1, _H) - bn_rm.reshape(1, _H) * scale)
    wa = jnp.stack([W_att[:_H, 0], W_att[_H:, 0]], axis=1)
    bvec = jnp.stack([b_att[0], jnp.zeros((), _f32)]).reshape(1, 2)
    emb, emb2, s12 = _tc_post(h, scale, shift, wa, bvec)
    embT = emb2.reshape(_NSC * _NT, 32)

    epad = jnp.concatenate(
        [edges, jnp.zeros((2, _EPPAD - _EP), _i32)], axis=1)
    e0 = epad[0].reshape(_EPPAD // 128, 128)
    e1 = epad[1].reshape(_EPPAD // 128, 128)
    eg = _egather(embT, e0, e1)
    pred_edges = _tc_edge_mlp(eg, W_ep1, b_ep1, W_ep2, b_ep2)

    dst4, src4 = prep_adj(end_day + 1)
    s1 = s12[:, 0]
    s2 = s12[:, 1]
    neigh, deg = _aggr(s1, s2, embT, dst4, src4)
    pred_nodes = _tc_node_mlp(emb, neigh.reshape(2, _NT, 32),
                              deg.reshape(_NT, 1), W_np1, b_np1,
                              W_np2, b_np2)
    return (pred_edges, pred_nodes)


# speedup vs baseline: 6.2151x; 1.1702x over previous
import functools

import jax
import jax.numpy as jnp
from jax import lax
from jax.experimental import pallas as pl
from jax.experimental.pallas import tpu as pltpu
from jax.experimental.pallas import tpu_sc as plsc

_N = 50000
_H = 64
_E = 800000
_EP = 100000
_NSC = 2
_NTILE = 16

_NT = ((_N + 1 + 127) // 128) * 128
_RPT = _NT // _NTILE
_TE = ((_E + _NTILE * 1024 - 1) // (_NTILE * 1024)) * 1024
_EPAD = _NTILE * _TE
_TEP = ((_EP + _NTILE * 256 - 1) // (_NTILE * 256)) * 256
_EPPAD = _NTILE * _TEP
_SENT = _N

_f32 = jnp.float32
_i32 = jnp.int32


def _mesh():
    return plsc.VectorSubcoreMesh(core_axis_name="c", subcore_axis_name="s")


_SC_PARAMS = pltpu.CompilerParams(use_tc_tiling_on_sc=False,
                                  needs_layout_passes=False)


def _zero_zbuf(zbuf):
    z = jnp.zeros((16,), _f32)

    def body(r, _):
        zbuf[r, pl.ds(0, 16)] = z
        zbuf[r, pl.ds(16, 16)] = z
        return 0

    lax.fori_loop(0, 128, body, 0)


def _zero_accum(accum, zbuf, s):
    base = s * _RPT
    nfull = _RPT // 128
    rem = _RPT % 128

    def body(k, _):
        pltpu.sync_copy(zbuf, accum.at[pl.ds(base + k * 128, 128)])
        return 0

    lax.fori_loop(0, nfull, body, 0)
    if rem:
        pltpu.sync_copy(zbuf.at[pl.ds(0, rem)],
                        accum.at[pl.ds(base + nfull * 128, rem)])


def _add_base(idx, nrows, cbase):
    for j in range(nrows):
        for t in range(8):
            sl = (j, pl.ds(t * 16, 16))
            idx[sl] = idx[sl] + cbase


def _spmm_body(table, dst2d, src2d, out, accum, idxs, idxd, rows, zbuf,
               sem_g, sem_s):
    c = lax.axis_index("c")
    s = lax.axis_index("s")
    _zero_zbuf(zbuf)
    _zero_accum(accum, zbuf, s)
    plsc.subcore_barrier()
    cbase = c * _NT
    rows_per_tile = _TE // 128

    def outer(k, _):
        r0 = s * rows_per_tile + k * 8
        pltpu.sync_copy(src2d.at[pl.ds(r0, 8)], idxs)
        pltpu.sync_copy(dst2d.at[pl.ds(r0, 8)], idxd)
        _add_base(idxs, 8, cbase)
        scat = [None, None]
        g = pltpu.async_copy(table.at[idxs.at[0]], rows.at[0], sem_g)
        for j in range(8):
            b = j & 1
            nb = 1 - b
            if j < 7:
                if scat[nb] is not None:
                    scat[nb].wait()
                g_next = pltpu.async_copy(table.at[idxs.at[j + 1]],
                                          rows.at[nb], sem_g)
            g.wait()
            scat[b] = pltpu.async_copy(rows.at[b], accum.at[idxd.at[j]],
                                       sem_s, add=True)
            if j < 7:
                g = g_next
        scat[0].wait()
        scat[1].wait()
        return 0

    lax.fori_loop(0, rows_per_tile // 8, outer, 0)
    plsc.subcore_barrier()
    wbase = s * _RPT
    pltpu.sync_copy(accum.at[pl.ds(wbase, _RPT)],
                    out.at[pl.ds(cbase + wbase, _RPT)])


_spmm = functools.partial(
    pl.kernel,
    out_type=jax.ShapeDtypeStruct((_NSC * _NT, 32), _f32),
    mesh=_mesh(),
    compiler_params=_SC_PARAMS,
    scratch_types=[
        pltpu.VMEM_SHARED((_NT, 32), _f32),
        pltpu.VMEM((8, 128), _i32),
        pltpu.VMEM((8, 128), _i32),
        pltpu.VMEM((2, 128, 32), _f32),
        pltpu.VMEM((128, 32), _f32),
        pltpu.SemaphoreType.DMA,
        pltpu.SemaphoreType.DMA,
    ],
)(_spmm_body)


def _aggr_body(s1h, s2h, table, dst2d, src2d, neigh_out, deg_out,
               nacc, dacc, a1buf, a2buf, idxd, idxs, rows, wbuf, degbuf,
               zbuf, dzero, sem, sem_g, sem_s, sem_d):
    c = lax.axis_index("c")
    s = lax.axis_index("s")
    _zero_zbuf(zbuf)
    z16 = jnp.zeros((16,), _f32)
    for t in range(8):
        dzero[pl.ds(t * 16, 16)] = z16
    _zero_accum(nacc, zbuf, s)
    base = s * _RPT
    nfull = _RPT // 128
    rem = _RPT % 128

    def zb(k, _):
        pltpu.sync_copy(dzero, dacc.at[pl.ds(base + k * 128, 128)])
        return 0

    lax.fori_loop(0, nfull, zb, 0)
    if rem:
        pltpu.sync_copy(dzero.at[pl.ds(0, rem)],
                        dacc.at[pl.ds(base + nfull * 128, rem)])
    plsc.subcore_barrier()
    cbase = c * _NT
    rows_per_tile = _TE // 128

    def outer(k, _):
        r0 = s * rows_per_tile + k * 8
        pltpu.sync_copy(dst2d.at[pl.ds(r0, 8)], idxd)
        pltpu.sync_copy(src2d.at[pl.ds(r0, 8)], idxs)
        descs = [pltpu.async_copy(s1h.at[idxd.at[j]], a1buf.at[j], sem)
                 for j in range(8)]
        descs += [pltpu.async_copy(s2h.at[idxs.at[j]], a2buf.at[j], sem)
                  for j in range(8)]
        for d in descs:
            d.wait()
        for j in range(8):
            for t in range(8):
                d16 = idxd[j, pl.ds(t * 16, 16)]
                s16 = idxs[j, pl.ds(t * 16, 16)]
                a1 = a1buf[j, pl.ds(t * 16, 16)]
                a2 = a2buf[j, pl.ds(t * 16, 16)]
                w = 1.0 / (1.0 + jnp.exp(-(a1 + a2)))
                m = d16 != s16
                wbuf[j, pl.ds(t * 16, 16)] = jnp.where(m, w, 0.0)
                degbuf[j, pl.ds(t * 16, 16)] = jnp.where(m, 1.0, 0.0)

        @pl.when(c == 0)
        def _():
            dd = [pltpu.async_copy(degbuf.at[j], dacc.at[idxd.at[j]],
                                   sem_d, add=True) for j in range(8)]
            for d in dd:
                d.wait()

        _add_base(idxs, 8, cbase)
        scat = [None, None]
        g = pltpu.async_copy(table.at[idxs.at[0]], rows.at[0], sem_g)
        for j in range(8):
            b = j & 1
            nb = 1 - b
            if j < 7:
                if scat[nb] is not None:
                    scat[nb].wait()
                g_next = pltpu.async_copy(table.at[idxs.at[j + 1]],
                                          rows.at[nb], sem_g)
            g.wait()

            def scale(r, _, j=j, b=b):
                bc = plsc.load_gather(
                    wbuf, [jnp.full((16,), j, _i32), jnp.full((16,), r, _i32)])
                rows[b, r, pl.ds(0, 16)] = rows[b, r, pl.ds(0, 16)] * bc
                rows[b, r, pl.ds(16, 16)] = rows[b, r, pl.ds(16, 16)] * bc
                return 0

            lax.fori_loop(0, 128, scale, 0)
            scat[b] = pltpu.async_copy(rows.at[b], nacc.at[idxd.at[j]],
                                       sem_s, add=True)
            if j < 7:
                g = g_next
        scat[0].wait()
        scat[1].wait()
        return 0

    lax.fori_loop(0, rows_per_tile // 8, outer, 0)
    plsc.subcore_barrier()
    wbase = s * _RPT
    pltpu.sync_copy(nacc.at[pl.ds(wbase, _RPT)],
                    neigh_out.at[pl.ds(cbase + wbase, _RPT)])

    @pl.when(c == 0)
    def _():
        pltpu.sync_copy(dacc.at[pl.ds(wbase, _RPT)],
                        deg_out.at[pl.ds(wbase, _RPT)])


_aggr = functools.partial(
    pl.kernel,
    out_type=(jax.ShapeDtypeStruct((_NSC * _NT, 32), _f32),
              jax.ShapeDtypeStruct((_NT,), _f32)),
    mesh=_mesh(),
    compiler_params=_SC_PARAMS,
    scratch_types=[
        pltpu.VMEM_SHARED((_NT, 32), _f32),
        pltpu.VMEM_SHARED((_NT,), _f32),
        pltpu.VMEM((8, 128), _f32),
        pltpu.VMEM((8, 128), _f32),
        pltpu.VMEM((8, 128), _i32),
        pltpu.VMEM((8, 128), _i32),
        pltpu.VMEM((2, 128, 32), _f32),
        pltpu.VMEM((8, 128), _f32),
        pltpu.VMEM((8, 128), _f32),
        pltpu.VMEM((128, 32), _f32),
        pltpu.VMEM((128,), _f32),
        pltpu.SemaphoreType.DMA,
        pltpu.SemaphoreType.DMA,
        pltpu.SemaphoreType.DMA,
        pltpu.SemaphoreType.DMA,
    ],
)(_aggr_body)


def _egather_body(table, e0_2d, e1_2d, out, idx, rows, sem):
    c = lax.axis_index("c")
    s = lax.axis_index("s")
    cbase = c * _NT
    rows_per_tile = _TEP // 128

    def outer(k, _):
        r0 = s * rows_per_tile + k * 2
        for j in range(2):
            earr = e0_2d if j == 0 else e1_2d
            pltpu.sync_copy(earr.at[pl.ds(r0, 2)], idx)
            _add_base(idx, 2, cbase)
            for t in range(2):
                pltpu.async_copy(table.at[idx.at[t]], rows, sem).wait()
                sec = (2 * j + c) * _EPPAD
                pltpu.sync_copy(
                    rows, out.at[pl.ds(sec + (r0 + t) * 128, 128)])
        return 0

    lax.fori_loop(0, rows_per_tile // 2, outer, 0)


_egather = functools.partial(
    pl.kernel,
    out_type=jax.ShapeDtypeStruct((4 * _EPPAD, 32), _f32),
    mesh=_mesh(),
    compiler_params=_SC_PARAMS,
    scratch_types=[
        pltpu.VMEM((2, 128), _i32),
        pltpu.VMEM((128, 32), _f32),
        pltpu.SemaphoreType.DMA,
    ],
)(_egather_body)


def _relu_mm_body(x2, b, w, o_ref):
    x = jnp.concatenate([x2[0], x2[1]], axis=1) + b[...]
    x = jnp.maximum(x, 0.0)
    sup = jnp.dot(x, w[...], preferred_element_type=_f32)
    o_ref[0] = sup[:, :32]
    o_ref[1] = sup[:, 32:]


def _tc_relu_mm(out1, b_gc0, W_gc1):
    bn = 1024
    grid = (_NT + bn - 1) // bn
    return pl.pallas_call(
        _relu_mm_body,
        grid=(grid,),
        in_specs=[
            pl.BlockSpec((2, bn, 32), lambda i: (0, i, 0)),
            pl.BlockSpec((1, _H), lambda i: (0, 0)),
            pl.BlockSpec((_H, _H), lambda i: (0, 0)),
        ],
        out_specs=pl.BlockSpec((2, bn, 32), lambda i: (0, i, 0)),
        out_shape=jax.ShapeDtypeStruct((2, _NT, 32), _f32),
    )(out1, b_gc0.reshape(1, _H), W_gc1)


def _gru_body(x2, b1, h, wih, whh, bih, bhh, o_ref):
    x = jnp.concatenate([x2[0], x2[1]], axis=1) + b1[...]
    gi = jnp.dot(x, wih[...], preferred_element_type=_f32) + bih[...]
    gh = jnp.dot(h[...], whh[...], preferred_element_type=_f32) + bhh[...]
    r = jax.nn.sigmoid(gi[:, :_H] + gh[:, :_H])
    z = jax.nn.sigmoid(gi[:, _H:2 * _H] + gh[:, _H:2 * _H])
    n = jnp.tanh(gi[:, 2 * _H:] + r * gh[:, 2 * _H:])
    o_ref[...] = (1.0 - z) * n + z * h[...]


def _tc_gru(out2, b_gc1, h, W_ihT, W_hhT, b_ih, b_hh):
    bn = 1000
    grid = _N // bn
    return pl.pallas_call(
        _gru_body,
        grid=(grid,),
        in_specs=[
            pl.BlockSpec((2, bn, 32), lambda i: (0, i, 0)),
            pl.BlockSpec((1, _H), lambda i: (0, 0)),
            pl.BlockSpec((bn, _H), lambda i: (i, 0)),
            pl.BlockSpec((_H, 3 * _H), lambda i: (0, 0)),
            pl.BlockSpec((_H, 3 * _H), lambda i: (0, 0)),
            pl.BlockSpec((1, 3 * _H), lambda i: (0, 0)),
            pl.BlockSpec((1, 3 * _H), lambda i: (0, 0)),
        ],
        out_specs=pl.BlockSpec((bn, _H), lambda i: (i, 0)),
        out_shape=jax.ShapeDtypeStruct((_N, _H), _f32),
    )(out2, b_gc1.reshape(1, _H), h, W_ihT, W_hhT,
      b_ih.reshape(1, 3 * _H), b_hh.reshape(1, 3 * _H))


def _post_body(h, scale, shift, wa, bvec, o_emb, o_emb2, o_s12):
    emb = h[...] * scale[...] + shift[...]
    o_emb[...] = emb
    o_emb2[0] = emb[:, :32]
    o_emb2[1] = emb[:, 32:]
    o_s12[...] = jnp.dot(emb, wa[...], preferred_element_type=_f32) + bvec[...]


def _tc_post(h, scale, shift, wa, bvec):
    bn = 1024
    grid = (_NT + bn - 1) // bn
    return pl.pallas_call(
        _post_body,
        grid=(grid,),
        in_specs=[
            pl.BlockSpec((bn, _H), lambda i: (i, 0)),
            pl.BlockSpec((1, _H), lambda i: (0, 0)),
            pl.BlockSpec((1, _H), lambda i: (0, 0)),
            pl.BlockSpec((_H, 2), lambda i: (0, 0)),
            pl.BlockSpec((1, 2), lambda i: (0, 0)),
        ],
        out_specs=[
            pl.BlockSpec((bn, _H), lambda i: (i, 0)),
            pl.BlockSpec((2, bn, 32), lambda i: (0, i, 0)),
            pl.BlockSpec((bn, 2), lambda i: (i, 0)),
        ],
        out_shape=[
            jax.ShapeDtypeStruct((_NT, _H), _f32),
            jax.ShapeDtypeStruct((2, _NT, 32), _f32),
            jax.ShapeDtypeStruct((_NT, 2), _f32),
        ],
    )(h, scale, shift, wa, bvec)


def _log_softmax2(lg):
    m = jnp.max(lg, axis=1, keepdims=True)
    ls = m + jnp.log(jnp.sum(jnp.exp(lg - m), axis=1, keepdims=True))
    return lg - ls


def _edge_mlp_body(nf0, nf1, ns0, ns1, w1, b1, w2, b2, o_ref):
    pe = (jnp.dot(nf0[...], w1[0:32, :], preferred_element_type=_f32)
          + jnp.dot(nf1[...], w1[32:64, :], preferred_element_type=_f32)
          + jnp.dot(ns0[...], w1[64:96, :], preferred_element_type=_f32)
          + jnp.dot(ns1[...], w1[96:128, :], preferred_element_type=_f32)
          + b1[...])
    pe = jnp.maximum(pe, 0.0)
    lg = jnp.dot(pe, w2[...], preferred_element_type=_f32) + b2[...]
    o_ref[...] = _log_softmax2(lg)


def _tc_edge_mlp(eg, W_ep1, b_ep1, W_ep2, b_ep2):
    bn = 512
    grid = (_EP + bn - 1) // bn
    nh = W_ep1.shape[1]
    sec = _EPPAD // bn

    def spec(k):
        return pl.BlockSpec((bn, 32), lambda i, k=k: (k * sec + i, 0))

    return pl.pallas_call(
        _edge_mlp_body,
        grid=(grid,),
        in_specs=[
            spec(0), spec(1), spec(2), spec(3),
            pl.BlockSpec((2 * _H, nh), lambda i: (0, 0)),
            pl.BlockSpec((1, nh), lambda i: (0, 0)),
            pl.BlockSpec((nh, 2), lambda i: (0, 0)),
            pl.BlockSpec((1, 2), lambda i: (0, 0)),
        ],
        out_specs=pl.BlockSpec((bn, 2), lambda i: (i, 0)),
        out_shape=jax.ShapeDtypeStruct((_EP, 2), _f32),
    )(eg, eg, eg, eg, W_ep1, b_ep1.reshape(1, nh), W_ep2,
      b_ep2.reshape(1, 2))


def _node_mlp_body(emb, n2, deg, w1, b1, w2, b2, o_ref):
    nb = jnp.concatenate([n2[0], n2[1]], axis=1)
    nb = nb / jnp.maximum(deg[...], 1.0)
    pn = (jnp.dot(emb[...], w1[0:_H, :], preferred_element_type=_f32)
          + jnp.dot(nb, w1[_H:2 * _H, :], preferred_element_type=_f32)
          + b1[...])
    pn = jnp.maximum(pn, 0.0)
    lg = jnp.dot(pn, w2[...], preferred_element_type=_f32) + b2[...]
    o_ref[...] = _log_softmax2(lg)


def _tc_node_mlp(emb, neigh, deg, W_np1, b_np1, W_np2, b_np2):
    bn = 1000
    grid = _N // bn
    nh = W_np1.shape[1]
    return pl.pallas_call(
        _node_mlp_body,
        grid=(grid,),
        in_specs=[
            pl.BlockSpec((bn, _H), lambda i: (i, 0)),
            pl.BlockSpec((2, bn, 32), lambda i: (0, i, 0)),
            pl.BlockSpec((bn, 1), lambda i: (i, 0)),
            pl.BlockSpec((2 * _H, nh), lambda i: (0, 0)),
            pl.BlockSpec((1, nh), lambda i: (0, 0)),
            pl.BlockSpec((nh, 2), lambda i: (0, 0)),
            pl.BlockSpec((1, 2), lambda i: (0, 0)),
        ],
        out_specs=pl.BlockSpec((bn, 2), lambda i: (i, 0)),
        out_shape=jax.ShapeDtypeStruct((_N, 2), _f32),
    )(emb, neigh, deg, W_np1, b_np1.reshape(1, nh), W_np2,
      b_np2.reshape(1, 2))


def kernel(adjs, edges, start_day, end_day, W_gc0, b_gc0, W_gc1, b_gc1,
           W_ih, W_hh, b_ih, b_hh, bn_gamma, bn_beta, bn_rm, bn_rv,
           W_ep1, b_ep1, W_ep2, b_ep2, W_np1, b_np1, W_np2, b_np2,
           W_att, b_att):
    adjs = adjs.astype(_i32)
    edges = edges.astype(_i32)
    nsnap = adjs.shape[0]

    def prep_adj(i):
        a = lax.dynamic_index_in_dim(adjs, i, 0, keepdims=False)
        pad = jnp.full((2, _EPAD - _E), _SENT, _i32)
        a = jnp.concatenate([a, pad], axis=1)
        return (a[0].reshape(_EPAD // 128, 128),
                a[1].reshape(_EPAD // 128, 128))

    T0 = jnp.zeros((_NSC * _NT, 32), _f32)
    T0 = T0.at[0:_N].set(W_gc0[:, :32]).at[_NT:_NT + _N].set(W_gc0[:, 32:])

    h = jnp.zeros((_N, _H), _f32)
    W_ihT = W_ih.T
    W_hhT = W_hh.T
    for i in range(nsnap - 1):
        dst, src = prep_adj(start_day + i)
        out1 = _spmm(T0, dst, src)
        sup2 = _tc_relu_mm(out1.reshape(2, _NT, 32), b_gc0, W_gc1)
        out2 = _spmm(sup2.reshape(_NSC * _NT, 32), dst, src)
        h = _tc_gru(out2.reshape(2, _NT, 32), b_gc1, h, W_ihT, W_hhT,
                    b_ih, b_hh)

    scale = (bn_gamma / jnp.sqrt(bn_rv + 1e-5)).reshape(1, _H)
    shift = (bn_beta.reshape(1, _H) - bn_rm.reshape(1, _H) * scale)
    wa = jnp.stack([W_att[:_H, 0], W_att[_H:, 0]], axis=1)
    bvec = jnp.stack([b_att[0], jnp.zeros((), _f32)]).reshape(1, 2)
    emb, emb2, s12 = _tc_post(h, scale, shift, wa, bvec)
    embT = emb2.reshape(_NSC * _NT, 32)

    epad = jnp.concatenate(
        [edges, jnp.zeros((2, _EPPAD - _EP), _i32)], axis=1)
    e0 = epad[0].reshape(_EPPAD // 128, 128)
    e1 = epad[1].reshape(_EPPAD // 128, 128)
    eg = _egather(embT, e0, e1)
    pred_edges = _tc_edge_mlp(eg, W_ep1, b_ep1, W_ep2, b_ep2)

    dst4, src4 = prep_adj(end_day + 1)
    s1 = s12[:, 0]
    s2 = s12[:, 1]
    neigh, deg = _aggr(s1, s2, embT, dst4, src4)
    pred_nodes = _tc_node_mlp(emb, neigh.reshape(2, _NT, 32),
                              deg.reshape(_NT, 1), W_np1, b_np1,
                              W_np2, b_np2)
    return (pred_edges, pred_nodes)


# speedup vs baseline: 7.0636x; 1.1365x over previous
import functools

import jax
import jax.numpy as jnp
from jax import lax
from jax.experimental import pallas as pl
from jax.experimental.pallas import tpu as pltpu
from jax.experimental.pallas import tpu_sc as plsc

_N = 50000
_H = 64
_E = 800000
_EP = 100000
_NSC = 2
_NTILE = 16

_NT = ((_N + 1 + 127) // 128) * 128
_RPT = _NT // _NTILE
_TE = ((_E + _NTILE * 1024 - 1) // (_NTILE * 1024)) * 1024
_EPAD = _NTILE * _TE
_TEP = ((_EP + _NTILE * 256 - 1) // (_NTILE * 256)) * 256
_EPPAD = _NTILE * _TEP
_SENT = _N

_f32 = jnp.float32
_i32 = jnp.int32


def _mesh():
    return plsc.VectorSubcoreMesh(core_axis_name="c", subcore_axis_name="s")


_SC_PARAMS = pltpu.CompilerParams(use_tc_tiling_on_sc=False,
                                  needs_layout_passes=False)


def _zero_zbuf(zbuf):
    z = jnp.zeros((16,), _f32)

    def body(r, _):
        zbuf[r, pl.ds(0, 16)] = z
        zbuf[r, pl.ds(16, 16)] = z
        return 0

    lax.fori_loop(0, 128, body, 0)


def _zero_accum(accum, zbuf, s):
    base = s * _RPT
    nfull = _RPT // 128
    rem = _RPT % 128

    def body(k, _):
        pltpu.sync_copy(zbuf, accum.at[pl.ds(base + k * 128, 128)])
        return 0

    lax.fori_loop(0, nfull, body, 0)
    if rem:
        pltpu.sync_copy(zbuf.at[pl.ds(0, rem)],
                        accum.at[pl.ds(base + nfull * 128, rem)])


def _add_base(idx, nrows, cbase):
    for j in range(nrows):
        for t in range(8):
            sl = (j, pl.ds(t * 16, 16))
            idx[sl] = idx[sl] + cbase


def _spmm_body(table, dst2d, src2d, out, accum, idxs, idxd, rows, zbuf,
               sem_g, sem_s):
    c = lax.axis_index("c")
    s = lax.axis_index("s")
    _zero_zbuf(zbuf)
    _zero_accum(accum, zbuf, s)
    plsc.subcore_barrier()
    cbase = c * _NT
    rows_per_tile = _TE // 128

    def outer(k, _):
        r0 = s * rows_per_tile + k * 8
        pltpu.sync_copy(src2d.at[pl.ds(r0, 8)], idxs)
        pltpu.sync_copy(dst2d.at[pl.ds(r0, 8)], idxd)
        _add_base(idxs, 8, cbase)
        scat = [None] * 4
        g = [None] * 4
        for j in range(3):
            g[j] = pltpu.async_copy(table.at[idxs.at[j]], rows.at[j], sem_g)
        for j in range(8):
            b = j & 3
            nj = j + 3
            if nj < 8:
                pb = nj & 3
                if scat[pb] is not None:
                    scat[pb].wait()
                g[pb] = pltpu.async_copy(table.at[idxs.at[nj]],
                                         rows.at[pb], sem_g)
            g[b].wait()
            scat[b] = pltpu.async_copy(rows.at[b], accum.at[idxd.at[j]],
                                       sem_s, add=True)
        for b in range(4):
            scat[b].wait()
        return 0

    lax.fori_loop(0, rows_per_tile // 8, outer, 0)
    plsc.subcore_barrier()
    wbase = s * _RPT
    pltpu.sync_copy(accum.at[pl.ds(wbase, _RPT)],
                    out.at[pl.ds(cbase + wbase, _RPT)])


_spmm = functools.partial(
    pl.kernel,
    out_type=jax.ShapeDtypeStruct((_NSC * _NT, 32), _f32),
    mesh=_mesh(),
    compiler_params=_SC_PARAMS,
    scratch_types=[
        pltpu.VMEM_SHARED((_NT, 32), _f32),
        pltpu.VMEM((8, 128), _i32),
        pltpu.VMEM((8, 128), _i32),
        pltpu.VMEM((4, 128, 32), _f32),
        pltpu.VMEM((128, 32), _f32),
        pltpu.SemaphoreType.DMA,
        pltpu.SemaphoreType.DMA,
    ],
)(_spmm_body)


def _aggr_body(s1h, s2h, table, dst2d, src2d, neigh_out, deg_out,
               nacc, dacc, a1buf, a2buf, idxd, idxs, rows, wbuf, degbuf,
               zbuf, dzero, sem, sem_g, sem_s, sem_d):
    c = lax.axis_index("c")
    s = lax.axis_index("s")
    _zero_zbuf(zbuf)
    z16 = jnp.zeros((16,), _f32)
    for t in range(8):
        dzero[pl.ds(t * 16, 16)] = z16
    _zero_accum(nacc, zbuf, s)
    base = s * _RPT
    nfull = _RPT // 128
    rem = _RPT % 128

    def zb(k, _):
        pltpu.sync_copy(dzero, dacc.at[pl.ds(base + k * 128, 128)])
        return 0

    lax.fori_loop(0, nfull, zb, 0)
    if rem:
        pltpu.sync_copy(dzero.at[pl.ds(0, rem)],
                        dacc.at[pl.ds(base + nfull * 128, rem)])
    plsc.subcore_barrier()
    cbase = c * _NT
    rows_per_tile = _TE // 128

    def outer(k, _):
        r0 = s * rows_per_tile + k * 8
        pltpu.sync_copy(dst2d.at[pl.ds(r0, 8)], idxd)
        pltpu.sync_copy(src2d.at[pl.ds(r0, 8)], idxs)
        descs = [pltpu.async_copy(s1h.at[idxd.at[j]], a1buf.at[j], sem)
                 for j in range(8)]
        descs += [pltpu.async_copy(s2h.at[idxs.at[j]], a2buf.at[j], sem)
                  for j in range(8)]
        for d in descs:
            d.wait()
        for j in range(8):
            for t in range(8):
                d16 = idxd[j, pl.ds(t * 16, 16)]
                s16 = idxs[j, pl.ds(t * 16, 16)]
                a1 = a1buf[j, pl.ds(t * 16, 16)]
                a2 = a2buf[j, pl.ds(t * 16, 16)]
                w = 1.0 / (1.0 + jnp.exp(-(a1 + a2)))
                m = d16 != s16
                wbuf[j, pl.ds(t * 16, 16)] = jnp.where(m, w, 0.0)
                degbuf[j, pl.ds(t * 16, 16)] = jnp.where(m, 1.0, 0.0)

        @pl.when(c == 0)
        def _():
            dd = [pltpu.async_copy(degbuf.at[j], dacc.at[idxd.at[j]],
                                   sem_d, add=True) for j in range(8)]
            for d in dd:
                d.wait()

        _add_base(idxs, 8, cbase)
        scat = [None, None]
        g = pltpu.async_copy(table.at[idxs.at[0]], rows.at[0], sem_g)
        for j in range(8):
            b = j & 1
            nb = 1 - b
            if j < 7:
                if scat[nb] is not None:
                    scat[nb].wait()
                g_next = pltpu.async_copy(table.at[idxs.at[j + 1]],
                                          rows.at[nb], sem_g)
            g.wait()

            def scale(r, _, j=j, b=b):
                bc = plsc.load_gather(
                    wbuf, [jnp.full((16,), j, _i32), jnp.full((16,), r, _i32)])
                rows[b, r, pl.ds(0, 16)] = rows[b, r, pl.ds(0, 16)] * bc
                rows[b, r, pl.ds(16, 16)] = rows[b, r, pl.ds(16, 16)] * bc
                return 0

            lax.fori_loop(0, 128, scale, 0)
            scat[b] = pltpu.async_copy(rows.at[b], nacc.at[idxd.at[j]],
                                       sem_s, add=True)
            if j < 7:
                g = g_next
        scat[0].wait()
        scat[1].wait()
        return 0

    lax.fori_loop(0, rows_per_tile // 8, outer, 0)
    plsc.subcore_barrier()
    wbase = s * _RPT
    pltpu.sync_copy(nacc.at[pl.ds(wbase, _RPT)],
                    neigh_out.at[pl.ds(cbase + wbase, _RPT)])

    @pl.when(c == 0)
    def _():
        pltpu.sync_copy(dacc.at[pl.ds(wbase, _RPT)],
                        deg_out.at[pl.ds(wbase, _RPT)])


_aggr = functools.partial(
    pl.kernel,
    out_type=(jax.ShapeDtypeStruct((_NSC * _NT, 32), _f32),
              jax.ShapeDtypeStruct((_NT,), _f32)),
    mesh=_mesh(),
    compiler_params=_SC_PARAMS,
    scratch_types=[
        pltpu.VMEM_SHARED((_NT, 32), _f32),
        pltpu.VMEM_SHARED((_NT,), _f32),
        pltpu.VMEM((8, 128), _f32),
        pltpu.VMEM((8, 128), _f32),
        pltpu.VMEM((8, 128), _i32),
        pltpu.VMEM((8, 128), _i32),
        pltpu.VMEM((2, 128, 32), _f32),
        pltpu.VMEM((8, 128), _f32),
        pltpu.VMEM((8, 128), _f32),
        pltpu.VMEM((128, 32), _f32),
        pltpu.VMEM((128,), _f32),
        pltpu.SemaphoreType.DMA,
        pltpu.SemaphoreType.DMA,
        pltpu.SemaphoreType.DMA,
        pltpu.SemaphoreType.DMA,
    ],
)(_aggr_body)


def _egather_body(table, e0_2d, e1_2d, out, idx, rows, sem):
    c = lax.axis_index("c")
    s = lax.axis_index("s")
    cbase = c * _NT
    rows_per_tile = _TEP // 128

    def outer(k, _):
        r0 = s * rows_per_tile + k * 2
        for j in range(2):
            earr = e0_2d if j == 0 else e1_2d
            pltpu.sync_copy(earr.at[pl.ds(r0, 2)], idx)
            _add_base(idx, 2, cbase)
            for t in range(2):
                pltpu.async_copy(table.at[idx.at[t]], rows, sem).wait()
                sec = (2 * j + c) * _EPPAD
                pltpu.sync_copy(
                    rows, out.at[pl.ds(sec + (r0 + t) * 128, 128)])
        return 0

    lax.fori_loop(0, rows_per_tile // 2, outer, 0)


_egather = functools.partial(
    pl.kernel,
    out_type=jax.ShapeDtypeStruct((4 * _EPPAD, 32), _f32),
    mesh=_mesh(),
    compiler_params=_SC_PARAMS,
    scratch_types=[
        pltpu.VMEM((2, 128), _i32),
        pltpu.VMEM((128, 32), _f32),
        pltpu.SemaphoreType.DMA,
    ],
)(_egather_body)


def _relu_mm_body(x2, b, w, o_ref):
    x = jnp.concatenate([x2[0], x2[1]], axis=1) + b[...]
    x = jnp.maximum(x, 0.0)
    sup = jnp.dot(x, w[...], preferred_element_type=_f32)
    o_ref[0] = sup[:, :32]
    o_ref[1] = sup[:, 32:]


def _tc_relu_mm(out1, b_gc0, W_gc1):
    bn = 1024
    grid = (_NT + bn - 1) // bn
    return pl.pallas_call(
        _relu_mm_body,
        grid=(grid,),
        in_specs=[
            pl.BlockSpec((2, bn, 32), lambda i: (0, i, 0)),
            pl.BlockSpec((1, _H), lambda i: (0, 0)),
            pl.BlockSpec((_H, _H), lambda i: (0, 0)),
        ],
        out_specs=pl.BlockSpec((2, bn, 32), lambda i: (0, i, 0)),
        out_shape=jax.ShapeDtypeStruct((2, _NT, 32), _f32),
    )(out1, b_gc0.reshape(1, _H), W_gc1)


def _gru_body(x2, b1, h, wih, whh, bih, bhh, o_ref):
    x = jnp.concatenate([x2[0], x2[1]], axis=1) + b1[...]
    gi = jnp.dot(x, wih[...], preferred_element_type=_f32) + bih[...]
    gh = jnp.dot(h[...], whh[...], preferred_element_type=_f32) + bhh[...]
    r = jax.nn.sigmoid(gi[:, :_H] + gh[:, :_H])
    z = jax.nn.sigmoid(gi[:, _H:2 * _H] + gh[:, _H:2 * _H])
    n = jnp.tanh(gi[:, 2 * _H:] + r * gh[:, 2 * _H:])
    o_ref[...] = (1.0 - z) * n + z * h[...]


def _tc_gru(out2, b_gc1, h, W_ihT, W_hhT, b_ih, b_hh):
    bn = 1000
    grid = _N // bn
    return pl.pallas_call(
        _gru_body,
        grid=(grid,),
        in_specs=[
            pl.BlockSpec((2, bn, 32), lambda i: (0, i, 0)),
            pl.BlockSpec((1, _H), lambda i: (0, 0)),
            pl.BlockSpec((bn, _H), lambda i: (i, 0)),
            pl.BlockSpec((_H, 3 * _H), lambda i: (0, 0)),
            pl.BlockSpec((_H, 3 * _H), lambda i: (0, 0)),
            pl.BlockSpec((1, 3 * _H), lambda i: (0, 0)),
            pl.BlockSpec((1, 3 * _H), lambda i: (0, 0)),
        ],
        out_specs=pl.BlockSpec((bn, _H), lambda i: (i, 0)),
        out_shape=jax.ShapeDtypeStruct((_N, _H), _f32),
    )(out2, b_gc1.reshape(1, _H), h, W_ihT, W_hhT,
      b_ih.reshape(1, 3 * _H), b_hh.reshape(1, 3 * _H))


def _post_body(h, scale, shift, wa, bvec, o_emb, o_emb2, o_s12):
    emb = h[...] * scale[...] + shift[...]
    o_emb[...] = emb
    o_emb2[0] = emb[:, :32]
    o_emb2[1] = emb[:, 32:]
    o_s12[...] = jnp.dot(emb, wa[...], preferred_element_type=_f32) + bvec[...]


def _tc_post(h, scale, shift, wa, bvec):
    bn = 1024
    grid = (_NT + bn - 1) // bn
    return pl.pallas_call(
        _post_body,
        grid=(grid,),
        in_specs=[
            pl.BlockSpec((bn, _H), lambda i: (i, 0)),
            pl.BlockSpec((1, _H), lambda i: (0, 0)),
            pl.BlockSpec((1, _H), lambda i: (0, 0)),
            pl.BlockSpec((_H, 2), lambda i: (0, 0)),
            pl.BlockSpec((1, 2), lambda i: (0, 0)),
        ],
        out_specs=[
            pl.BlockSpec((bn, _H), lambda i: (i, 0)),
            pl.BlockSpec((2, bn, 32), lambda i: (0, i, 0)),
            pl.BlockSpec((bn, 2), lambda i: (i, 0)),
        ],
        out_shape=[
            jax.ShapeDtypeStruct((_NT, _H), _f32),
            jax.ShapeDtypeStruct((2, _NT, 32), _f32),
            jax.ShapeDtypeStruct((_NT, 2), _f32),
        ],
    )(h, scale, shift, wa, bvec)


def _log_softmax2(lg):
    m = jnp.max(lg, axis=1, keepdims=True)
    ls = m + jnp.log(jnp.sum(jnp.exp(lg - m), axis=1, keepdims=True))
    return lg - ls


def _edge_mlp_body(nf0, nf1, ns0, ns1, w1, b1, w2, b2, o_ref):
    pe = (jnp.dot(nf0[...], w1[0:32, :], preferred_element_type=_f32)
          + jnp.dot(nf1[...], w1[32:64, :], preferred_element_type=_f32)
          + jnp.dot(ns0[...], w1[64:96, :], preferred_element_type=_f32)
          + jnp.dot(ns1[...], w1[96:128, :], preferred_element_type=_f32)
          + b1[...])
    pe = jnp.maximum(pe, 0.0)
    lg = jnp.dot(pe, w2[...], preferred_element_type=_f32) + b2[...]
    o_ref[...] = _log_softmax2(lg)


def _tc_edge_mlp(eg, W_ep1, b_ep1, W_ep2, b_ep2):
    bn = 512
    grid = (_EP + bn - 1) // bn
    nh = W_ep1.shape[1]
    sec = _EPPAD // bn

    def spec(k):
        return pl.BlockSpec((bn, 32), lambda i, k=k: (k * sec + i, 0))

    return pl.pallas_call(
        _edge_mlp_body,
        grid=(grid,),
        in_specs=[
            spec(0), spec(1), spec(2), spec(3),
            pl.BlockSpec((2 * _H, nh), lambda i: (0, 0)),
            pl.BlockSpec((1, nh), lambda i: (0, 0)),
            pl.BlockSpec((nh, 2), lambda i: (0, 0)),
            pl.BlockSpec((1, 2), lambda i: (0, 0)),
        ],
        out_specs=pl.BlockSpec((bn, 2), lambda i: (i, 0)),
        out_shape=jax.ShapeDtypeStruct((_EP, 2), _f32),
    )(eg, eg, eg, eg, W_ep1, b_ep1.reshape(1, nh), W_ep2,
      b_ep2.reshape(1, 2))


def _node_mlp_body(emb, n2, deg, w1, b1, w2, b2, o_ref):
    nb = jnp.concatenate([n2[0], n2[1]], axis=1)
    nb = nb / jnp.maximum(deg[...], 1.0)
    pn = (jnp.dot(emb[...], w1[0:_H, :], preferred_element_type=_f32)
          + jnp.dot(nb, w1[_H:2 * _H, :], preferred_element_type=_f32)
          + b1[...])
    pn = jnp.maximum(pn, 0.0)
    lg = jnp.dot(pn, w2[...], preferred_element_type=_f32) + b2[...]
    o_ref[...] = _log_softmax2(lg)


def _tc_node_mlp(emb, neigh, deg, W_np1, b_np1, W_np2, b_np2):
    bn = 1000
    grid = _N // bn
    nh = W_np1.shape[1]
    return pl.pallas_call(
        _node_mlp_body,
        grid=(grid,),
        in_specs=[
            pl.BlockSpec((bn, _H), lambda i: (i, 0)),
            pl.BlockSpec((2, bn, 32), lambda i: (0, i, 0)),
            pl.BlockSpec((bn, 1), lambda i: (i, 0)),
            pl.BlockSpec((2 * _H, nh), lambda i: (0, 0)),
            pl.BlockSpec((1, nh), lambda i: (0, 0)),
            pl.BlockSpec((nh, 2), lambda i: (0, 0)),
            pl.BlockSpec((1, 2), lambda i: (0, 0)),
        ],
        out_specs=pl.BlockSpec((bn, 2), lambda i: (i, 0)),
        out_shape=jax.ShapeDtypeStruct((_N, 2), _f32),
    )(emb, neigh, deg, W_np1, b_np1.reshape(1, nh), W_np2,
      b_np2.reshape(1, 2))


def kernel(adjs, edges, start_day, end_day, W_gc0, b_gc0, W_gc1, b_gc1,
           W_ih, W_hh, b_ih, b_hh, bn_gamma, bn_beta, bn_rm, bn_rv,
           W_ep1, b_ep1, W_ep2, b_ep2, W_np1, b_np1, W_np2, b_np2,
           W_att, b_att):
    adjs = adjs.astype(_i32)
    edges = edges.astype(_i32)
    nsnap = adjs.shape[0]

    def prep_adj(i):
        a = lax.dynamic_index_in_dim(adjs, i, 0, keepdims=False)
        pad = jnp.full((2, _EPAD - _E), _SENT, _i32)
        a = jnp.concatenate([a, pad], axis=1)
        return (a[0].reshape(_EPAD // 128, 128),
                a[1].reshape(_EPAD // 128, 128))

    T0 = jnp.zeros((_NSC * _NT, 32), _f32)
    T0 = T0.at[0:_N].set(W_gc0[:, :32]).at[_NT:_NT + _N].set(W_gc0[:, 32:])

    h = jnp.zeros((_N, _H), _f32)
    W_ihT = W_ih.T
    W_hhT = W_hh.T
    for i in range(nsnap - 1):
        dst, src = prep_adj(start_day + i)
        out1 = _spmm(T0, dst, src)
        sup2 = _tc_relu_mm(out1.reshape(2, _NT, 32), b_gc0, W_gc1)
        out2 = _spmm(sup2.reshape(_NSC * _NT, 32), dst, src)
        h = _tc_gru(out2.reshape(2, _NT, 32), b_gc1, h, W_ihT, W_hhT,
                    b_ih, b_hh)

    scale = (bn_gamma / jnp.sqrt(bn_rv + 1e-5)).reshape(1, _H)
    shift = (bn_beta.reshape(1, _H) - bn_rm.reshape(1, _H) * scale)
    wa = jnp.stack([W_att[:_H, 0], W_att[_H:, 0]], axis=1)
    bvec = jnp.stack([b_att[0], jnp.zeros((), _f32)]).reshape(1, 2)
    emb, emb2, s12 = _tc_post(h, scale, shift, wa, bvec)
    embT = emb2.reshape(_NSC * _NT, 32)

    epad = jnp.concatenate(
        [edges, jnp.zeros((2, _EPPAD - _EP), _i32)], axis=1)
    e0 = epad[0].reshape(_EPPAD // 128, 128)
    e1 = epad[1].reshape(_EPPAD // 128, 128)
    eg = _egather(embT, e0, e1)
    pred_edges = _tc_edge_mlp(eg, W_ep1, b_ep1, W_ep2, b_ep2)

    dst4, src4 = prep_adj(end_day + 1)
    s1 = s12[:, 0]
    s2 = s12[:, 1]
    neigh, deg = _aggr(s1, s2, embT, dst4, src4)
    pred_nodes = _tc_node_mlp(emb, neigh.reshape(2, _NT, 32),
                              deg.reshape(_NT, 1), W_np1, b_np1,
                              W_np2, b_np2)
    return (pred_edges, pred_nodes)


# speedup vs baseline: 7.1101x; 1.0066x over previous
import functools

import jax
import jax.numpy as jnp
from jax import lax
from jax.experimental import pallas as pl
from jax.experimental.pallas import tpu as pltpu
from jax.experimental.pallas import tpu_sc as plsc

_N = 50000
_H = 64
_E = 800000
_EP = 100000
_NSC = 2
_NTILE = 16

_NT = ((_N + 1 + 127) // 128) * 128
_RPT = _NT // _NTILE
_TE = ((_E + _NTILE * 1024 - 1) // (_NTILE * 1024)) * 1024
_EPAD = _NTILE * _TE
_TEP = ((_EP + _NTILE * 256 - 1) // (_NTILE * 256)) * 256
_EPPAD = _NTILE * _TEP
_SENT = _N

_f32 = jnp.float32
_i32 = jnp.int32


def _mesh():
    return plsc.VectorSubcoreMesh(core_axis_name="c", subcore_axis_name="s")


_SC_PARAMS = pltpu.CompilerParams(use_tc_tiling_on_sc=False,
                                  needs_layout_passes=False)


def _zero_zbuf(zbuf):
    z = jnp.zeros((16,), _f32)

    def body(r, _):
        zbuf[r, pl.ds(0, 16)] = z
        zbuf[r, pl.ds(16, 16)] = z
        return 0

    lax.fori_loop(0, 128, body, 0)


def _zero_accum(accum, zbuf, s):
    base = s * _RPT
    nfull = _RPT // 128
    rem = _RPT % 128

    def body(k, _):
        pltpu.sync_copy(zbuf, accum.at[pl.ds(base + k * 128, 128)])
        return 0

    lax.fori_loop(0, nfull, body, 0)
    if rem:
        pltpu.sync_copy(zbuf.at[pl.ds(0, rem)],
                        accum.at[pl.ds(base + nfull * 128, rem)])


def _add_base(idx, nrows, cbase):
    for j in range(nrows):
        for t in range(8):
            sl = (j, pl.ds(t * 16, 16))
            idx[sl] = idx[sl] + cbase


def _spmm_body(table, dst2d, src2d, out, accum, idxs, idxd, rows, zbuf,
               sem_g, sem_s):
    c = lax.axis_index("c")
    s = lax.axis_index("s")
    _zero_zbuf(zbuf)
    _zero_accum(accum, zbuf, s)
    plsc.subcore_barrier()
    cbase = c * _NT
    rows_per_tile = _TE // 128

    def outer(k, _):
        r0 = s * rows_per_tile + k * 8
        pltpu.sync_copy(src2d.at[pl.ds(r0, 8)], idxs)
        pltpu.sync_copy(dst2d.at[pl.ds(r0, 8)], idxd)
        _add_base(idxs, 8, cbase)
        scat = [None] * 4
        g = [None] * 4
        for j in range(3):
            g[j] = pltpu.async_copy(table.at[idxs.at[j]], rows.at[j], sem_g)
        for j in range(8):
            b = j & 3
            nj = j + 3
            if nj < 8:
                pb = nj & 3
                if scat[pb] is not None:
                    scat[pb].wait()
                g[pb] = pltpu.async_copy(table.at[idxs.at[nj]],
                                         rows.at[pb], sem_g)
            g[b].wait()
            scat[b] = pltpu.async_copy(rows.at[b], accum.at[idxd.at[j]],
                                       sem_s, add=True)
        for b in range(4):
            scat[b].wait()
        return 0

    lax.fori_loop(0, rows_per_tile // 8, outer, 0)
    plsc.subcore_barrier()
    wbase = s * _RPT
    pltpu.sync_copy(accum.at[pl.ds(wbase, _RPT)],
                    out.at[pl.ds(cbase + wbase, _RPT)])


_spmm = functools.partial(
    pl.kernel,
    out_type=jax.ShapeDtypeStruct((_NSC * _NT, 32), _f32),
    mesh=_mesh(),
    compiler_params=_SC_PARAMS,
    scratch_types=[
        pltpu.VMEM_SHARED((_NT, 32), _f32),
        pltpu.VMEM((8, 128), _i32),
        pltpu.VMEM((8, 128), _i32),
        pltpu.VMEM((4, 128, 32), _f32),
        pltpu.VMEM((128, 32), _f32),
        pltpu.SemaphoreType.DMA,
        pltpu.SemaphoreType.DMA,
    ],
)(_spmm_body)


def _aggr_body(s1h, s2h, table, dst2d, src2d, neigh_out, deg_out,
               nacc, dacc, a1buf, a2buf, idxd, idxs, rows, wbuf, degbuf,
               zbuf, dzero, sem, sem_g, sem_s, sem_d):
    c = lax.axis_index("c")
    s = lax.axis_index("s")
    _zero_zbuf(zbuf)
    z16 = jnp.zeros((16,), _f32)
    for t in range(8):
        dzero[pl.ds(t * 16, 16)] = z16
    _zero_accum(nacc, zbuf, s)
    base = s * _RPT
    nfull = _RPT // 128
    rem = _RPT % 128

    def zb(k, _):
        pltpu.sync_copy(dzero, dacc.at[pl.ds(base + k * 128, 128)])
        return 0

    lax.fori_loop(0, nfull, zb, 0)
    if rem:
        pltpu.sync_copy(dzero.at[pl.ds(0, rem)],
                        dacc.at[pl.ds(base + nfull * 128, rem)])
    plsc.subcore_barrier()
    cbase = c * _NT
    rows_per_tile = _TE // 128

    def outer(k, _):
        r0 = s * rows_per_tile + k * 8
        pltpu.sync_copy(dst2d.at[pl.ds(r0, 8)], idxd)
        pltpu.sync_copy(src2d.at[pl.ds(r0, 8)], idxs)
        descs = [pltpu.async_copy(s1h.at[idxd.at[j]], a1buf.at[j], sem)
                 for j in range(8)]
        descs += [pltpu.async_copy(s2h.at[idxs.at[j]], a2buf.at[j], sem)
                  for j in range(8)]
        for d in descs:
            d.wait()
        for j in range(8):
            for t in range(8):
                d16 = idxd[j, pl.ds(t * 16, 16)]
                s16 = idxs[j, pl.ds(t * 16, 16)]
                a1 = a1buf[j, pl.ds(t * 16, 16)]
                a2 = a2buf[j, pl.ds(t * 16, 16)]
                w = 1.0 / (1.0 + jnp.exp(-(a1 + a2)))
                m = d16 != s16
                wbuf[j, pl.ds(t * 16, 16)] = jnp.where(m, w, 0.0)
                degbuf[j, pl.ds(t * 16, 16)] = jnp.where(m, 1.0, 0.0)

        @pl.when(c == 0)
        def _():
            dd = [pltpu.async_copy(degbuf.at[j], dacc.at[idxd.at[j]],
                                   sem_d, add=True) for j in range(8)]
            for d in dd:
                d.wait()

        _add_base(idxs, 8, cbase)
        scat = [None] * 4
        g = [None] * 4
        for j in range(3):
            g[j] = pltpu.async_copy(table.at[idxs.at[j]], rows.at[j], sem_g)
        for j in range(8):
            b = j & 3
            nj = j + 3
            if nj < 8:
                pb = nj & 3
                if scat[pb] is not None:
                    scat[pb].wait()
                g[pb] = pltpu.async_copy(table.at[idxs.at[nj]],
                                         rows.at[pb], sem_g)
            g[b].wait()

            def scale(r, _, j=j, b=b):
                bc = plsc.load_gather(
                    wbuf, [jnp.full((16,), j, _i32), jnp.full((16,), r, _i32)])
                rows[b, r, pl.ds(0, 16)] = rows[b, r, pl.ds(0, 16)] * bc
                rows[b, r, pl.ds(16, 16)] = rows[b, r, pl.ds(16, 16)] * bc
                return 0

            lax.fori_loop(0, 128, scale, 0)
            scat[b] = pltpu.async_copy(rows.at[b], nacc.at[idxd.at[j]],
                                       sem_s, add=True)
        for b in range(4):
            scat[b].wait()
        return 0

    lax.fori_loop(0, rows_per_tile // 8, outer, 0)
    plsc.subcore_barrier()
    wbase = s * _RPT
    pltpu.sync_copy(nacc.at[pl.ds(wbase, _RPT)],
                    neigh_out.at[pl.ds(cbase + wbase, _RPT)])

    @pl.when(c == 0)
    def _():
        pltpu.sync_copy(dacc.at[pl.ds(wbase, _RPT)],
                        deg_out.at[pl.ds(wbase, _RPT)])


_aggr = functools.partial(
    pl.kernel,
    out_type=(jax.ShapeDtypeStruct((_NSC * _NT, 32), _f32),
              jax.ShapeDtypeStruct((_NT,), _f32)),
    mesh=_mesh(),
    compiler_params=_SC_PARAMS,
    scratch_types=[
        pltpu.VMEM_SHARED((_NT, 32), _f32),
        pltpu.VMEM_SHARED((_NT,), _f32),
        pltpu.VMEM((8, 128), _f32),
        pltpu.VMEM((8, 128), _f32),
        pltpu.VMEM((8, 128), _i32),
        pltpu.VMEM((8, 128), _i32),
        pltpu.VMEM((4, 128, 32), _f32),
        pltpu.VMEM((8, 128), _f32),
        pltpu.VMEM((8, 128), _f32),
        pltpu.VMEM((128, 32), _f32),
        pltpu.VMEM((128,), _f32),
        pltpu.SemaphoreType.DMA,
        pltpu.SemaphoreType.DMA,
        pltpu.SemaphoreType.DMA,
        pltpu.SemaphoreType.DMA,
    ],
)(_aggr_body)


def _egather_body(table, e0_2d, e1_2d, out, idx, rows, sem):
    c = lax.axis_index("c")
    s = lax.axis_index("s")
    cbase = c * _NT
    rows_per_tile = _TEP // 128

    def outer(k, _):
        r0 = s * rows_per_tile + k * 2
        for j in range(2):
            earr = e0_2d if j == 0 else e1_2d
            pltpu.sync_copy(earr.at[pl.ds(r0, 2)], idx)
            _add_base(idx, 2, cbase)
            for t in range(2):
                pltpu.async_copy(table.at[idx.at[t]], rows, sem).wait()
                sec = (2 * j + c) * _EPPAD
                pltpu.sync_copy(
                    rows, out.at[pl.ds(sec + (r0 + t) * 128, 128)])
        return 0

    lax.fori_loop(0, rows_per_tile // 2, outer, 0)


_egather = functools.partial(
    pl.kernel,
    out_type=jax.ShapeDtypeStruct((4 * _EPPAD, 32), _f32),
    mesh=_mesh(),
    compiler_params=_SC_PARAMS,
    scratch_types=[
        pltpu.VMEM((2, 128), _i32),
        pltpu.VMEM((128, 32), _f32),
        pltpu.SemaphoreType.DMA,
    ],
)(_egather_body)


def _relu_mm_body(x2, b, w, o_ref):
    x = jnp.concatenate([x2[0], x2[1]], axis=1) + b[...]
    x = jnp.maximum(x, 0.0)
    sup = jnp.dot(x, w[...], preferred_element_type=_f32)
    o_ref[0] = sup[:, :32]
    o_ref[1] = sup[:, 32:]


def _tc_relu_mm(out1, b_gc0, W_gc1):
    bn = 1024
    grid = (_NT + bn - 1) // bn
    return pl.pallas_call(
        _relu_mm_body,
        grid=(grid,),
        in_specs=[
            pl.BlockSpec((2, bn, 32), lambda i: (0, i, 0)),
            pl.BlockSpec((1, _H), lambda i: (0, 0)),
            pl.BlockSpec((_H, _H), lambda i: (0, 0)),
        ],
        out_specs=pl.BlockSpec((2, bn, 32), lambda i: (0, i, 0)),
        out_shape=jax.ShapeDtypeStruct((2, _NT, 32), _f32),
    )(out1, b_gc0.reshape(1, _H), W_gc1)


def _gru_body(x2, b1, h, wih, whh, bih, bhh, o_ref):
    x = jnp.concatenate([x2[0], x2[1]], axis=1) + b1[...]
    gi = jnp.dot(x, wih[...], preferred_element_type=_f32) + bih[...]
    gh = jnp.dot(h[...], whh[...], preferred_element_type=_f32) + bhh[...]
    r = jax.nn.sigmoid(gi[:, :_H] + gh[:, :_H])
    z = jax.nn.sigmoid(gi[:, _H:2 * _H] + gh[:, _H:2 * _H])
    n = jnp.tanh(gi[:, 2 * _H:] + r * gh[:, 2 * _H:])
    o_ref[...] = (1.0 - z) * n + z * h[...]


def _tc_gru(out2, b_gc1, h, W_ihT, W_hhT, b_ih, b_hh):
    bn = 1000
    grid = _N // bn
    return pl.pallas_call(
        _gru_body,
        grid=(grid,),
        in_specs=[
            pl.BlockSpec((2, bn, 32), lambda i: (0, i, 0)),
            pl.BlockSpec((1, _H), lambda i: (0, 0)),
            pl.BlockSpec((bn, _H), lambda i: (i, 0)),
            pl.BlockSpec((_H, 3 * _H), lambda i: (0, 0)),
            pl.BlockSpec((_H, 3 * _H), lambda i: (0, 0)),
            pl.BlockSpec((1, 3 * _H), lambda i: (0, 0)),
            pl.BlockSpec((1, 3 * _H), lambda i: (0, 0)),
        ],
        out_specs=pl.BlockSpec((bn, _H), lambda i: (i, 0)),
        out_shape=jax.ShapeDtypeStruct((_N, _H), _f32),
    )(out2, b_gc1.reshape(1, _H), h, W_ihT, W_hhT,
      b_ih.reshape(1, 3 * _H), b_hh.reshape(1, 3 * _H))


def _post_body(h, scale, shift, wa, bvec, o_emb, o_emb2, o_s12):
    emb = h[...] * scale[...] + shift[...]
    o_emb[...] = emb
    o_emb2[0] = emb[:, :32]
    o_emb2[1] = emb[:, 32:]
    o_s12[...] = jnp.dot(emb, wa[...], preferred_element_type=_f32) + bvec[...]


def _tc_post(h, scale, shift, wa, bvec):
    bn = 1024
    grid = (_NT + bn - 1) // bn
    return pl.pallas_call(
        _post_body,
        grid=(grid,),
        in_specs=[
            pl.BlockSpec((bn, _H), lambda i: (i, 0)),
            pl.BlockSpec((1, _H), lambda i: (0, 0)),
            pl.BlockSpec((1, _H), lambda i: (0, 0)),
            pl.BlockSpec((_H, 2), lambda i: (0, 0)),
            pl.BlockSpec((1, 2), lambda i: (0, 0)),
        ],
        out_specs=[
            pl.BlockSpec((bn, _H), lambda i: (i, 0)),
            pl.BlockSpec((2, bn, 32), lambda i: (0, i, 0)),
            pl.BlockSpec((bn, 2), lambda i: (i, 0)),
        ],
        out_shape=[
            jax.ShapeDtypeStruct((_NT, _H), _f32),
            jax.ShapeDtypeStruct((2, _NT, 32), _f32),
            jax.ShapeDtypeStruct((_NT, 2), _f32),
        ],
    )(h, scale, shift, wa, bvec)


def _log_softmax2(lg):
    m = jnp.max(lg, axis=1, keepdims=True)
    ls = m + jnp.log(jnp.sum(jnp.exp(lg - m), axis=1, keepdims=True))
    return lg - ls


def _edge_mlp_body(nf0, nf1, ns0, ns1, w1, b1, w2, b2, o_ref):
    pe = (jnp.dot(nf0[...], w1[0:32, :], preferred_element_type=_f32)
          + jnp.dot(nf1[...], w1[32:64, :], preferred_element_type=_f32)
          + jnp.dot(ns0[...], w1[64:96, :], preferred_element_type=_f32)
          + jnp.dot(ns1[...], w1[96:128, :], preferred_element_type=_f32)
          + b1[...])
    pe = jnp.maximum(pe, 0.0)
    lg = jnp.dot(pe, w2[...], preferred_element_type=_f32) + b2[...]
    o_ref[...] = _log_softmax2(lg)


def _tc_edge_mlp(eg, W_ep1, b_ep1, W_ep2, b_ep2):
    bn = 512
    grid = (_EP + bn - 1) // bn
    nh = W_ep1.shape[1]
    sec = _EPPAD // bn

    def spec(k):
        return pl.BlockSpec((bn, 32), lambda i, k=k: (k * sec + i, 0))

    return pl.pallas_call(
        _edge_mlp_body,
        grid=(grid,),
        in_specs=[
            spec(0), spec(1), spec(2), spec(3),
            pl.BlockSpec((2 * _H, nh), lambda i: (0, 0)),
            pl.BlockSpec((1, nh), lambda i: (0, 0)),
            pl.BlockSpec((nh, 2), lambda i: (0, 0)),
            pl.BlockSpec((1, 2), lambda i: (0, 0)),
        ],
        out_specs=pl.BlockSpec((bn, 2), lambda i: (i, 0)),
        out_shape=jax.ShapeDtypeStruct((_EP, 2), _f32),
    )(eg, eg, eg, eg, W_ep1, b_ep1.reshape(1, nh), W_ep2,
      b_ep2.reshape(1, 2))


def _node_mlp_body(emb, n2, deg, w1, b1, w2, b2, o_ref):
    nb = jnp.concatenate([n2[0], n2[1]], axis=1)
    nb = nb / jnp.maximum(deg[...], 1.0)
    pn = (jnp.dot(emb[...], w1[0:_H, :], preferred_element_type=_f32)
          + jnp.dot(nb, w1[_H:2 * _H, :], preferred_element_type=_f32)
          + b1[...])
    pn = jnp.maximum(pn, 0.0)
    lg = jnp.dot(pn, w2[...], preferred_element_type=_f32) + b2[...]
    o_ref[...] = _log_softmax2(lg)


def _tc_node_mlp(emb, neigh, deg, W_np1, b_np1, W_np2, b_np2):
    bn = 1000
    grid = _N // bn
    nh = W_np1.shape[1]
    return pl.pallas_call(
        _node_mlp_body,
        grid=(grid,),
        in_specs=[
            pl.BlockSpec((bn, _H), lambda i: (i, 0)),
            pl.BlockSpec((2, bn, 32), lambda i: (0, i, 0)),
            pl.BlockSpec((bn, 1), lambda i: (i, 0)),
            pl.BlockSpec((2 * _H, nh), lambda i: (0, 0)),
            pl.BlockSpec((1, nh), lambda i: (0, 0)),
            pl.BlockSpec((nh, 2), lambda i: (0, 0)),
            pl.BlockSpec((1, 2), lambda i: (0, 0)),
        ],
        out_specs=pl.BlockSpec((bn, 2), lambda i: (i, 0)),
        out_shape=jax.ShapeDtypeStruct((_N, 2), _f32),
    )(emb, neigh, deg, W_np1, b_np1.reshape(1, nh), W_np2,
      b_np2.reshape(1, 2))


def kernel(adjs, edges, start_day, end_day, W_gc0, b_gc0, W_gc1, b_gc1,
           W_ih, W_hh, b_ih, b_hh, bn_gamma, bn_beta, bn_rm, bn_rv,
           W_ep1, b_ep1, W_ep2, b_ep2, W_np1, b_np1, W_np2, b_np2,
           W_att, b_att):
    adjs = adjs.astype(_i32)
    edges = edges.astype(_i32)
    nsnap = adjs.shape[0]

    def prep_adj(i):
        a = lax.dynamic_index_in_dim(adjs, i, 0, keepdims=False)
        pad = jnp.full((2, _EPAD - _E), _SENT, _i32)
        a = jnp.concatenate([a, pad], axis=1)
        return (a[0].reshape(_EPAD // 128, 128),
                a[1].reshape(_EPAD // 128, 128))

    T0 = jnp.zeros((_NSC * _NT, 32), _f32)
    T0 = T0.at[0:_N].set(W_gc0[:, :32]).at[_NT:_NT + _N].set(W_gc0[:, 32:])

    h = jnp.zeros((_N, _H), _f32)
    W_ihT = W_ih.T
    W_hhT = W_hh.T
    for i in range(nsnap - 1):
        dst, src = prep_adj(start_day + i)
        out1 = _spmm(T0, dst, src)
        sup2 = _tc_relu_mm(out1.reshape(2, _NT, 32), b_gc0, W_gc1)
        out2 = _spmm(sup2.reshape(_NSC * _NT, 32), dst, src)
        h = _tc_gru(out2.reshape(2, _NT, 32), b_gc1, h, W_ihT, W_hhT,
                    b_ih, b_hh)

    scale = (bn_gamma / jnp.sqrt(bn_rv + 1e-5)).reshape(1, _H)
    shift = (bn_beta.reshape(1, _H) - bn_rm.reshape(1, _H) * scale)
    wa = jnp.stack([W_att[:_H, 0], W_att[_H:, 0]], axis=1)
    bvec = jnp.stack([b_att[0], jnp.zeros((), _f32)]).reshape(1, 2)
    emb, emb2, s12 = _tc_post(h, scale, shift, wa, bvec)
    embT = emb2.reshape(_NSC * _NT, 32)

    epad = jnp.concatenate(
        [edges, jnp.zeros((2, _EPPAD - _EP), _i32)], axis=1)
    e0 = epad[0].reshape(_EPPAD // 128, 128)
    e1 = epad[1].reshape(_EPPAD // 128, 128)
    eg = _egather(embT, e0, e1)
    pred_edges = _tc_edge_mlp(eg, W_ep1, b_ep1, W_ep2, b_ep2)

    dst4, src4 = prep_adj(end_day + 1)
    s1 = s12[:, 0]
    s2 = s12[:, 1]
    neigh, deg = _aggr(s1, s2, embT, dst4, src4)
    pred_nodes = _tc_node_mlp(emb, neigh.reshape(2, _NT, 32),
                              deg.reshape(_NT, 1), W_np1, b_np1,
                              W_np2, b_np2)
    return (pred_edges, pred_nodes)
